# Initial kernel scaffold; baseline (speedup 1.0000x reference)
import jax
import jax.numpy as jnp
from jax.experimental import pallas as pl


def kernel(x, edge_index, edge_weight, W1, b1, W2, b2, W3, att_src, att_dst, b3):
    raise NotImplementedError("write your pallas kernel here")


# speedup vs baseline: 14.6170x; 14.6170x over previous
import dataclasses
import functools

import jax
import jax.numpy as jnp
from jax import lax
from jax.experimental import pallas as pl
from jax.experimental.pallas import tpu as pltpu
from jax.experimental.pallas import tpu_sc as plsc

N_NODES = 10000
N_EDGES = 320000
NP = 10240
N_WORKERS = 32
CH = 128
EPW = 10240
EPAD = EPW * N_WORKERS
ROWS_PER_TILE = NP // 16
CHUNKS_PER_TILE = ROWS_PER_TILE // CH

_MESH = plsc.VectorSubcoreMesh(core_axis_name="c", subcore_axis_name="s")

_SC_PARAMS = pltpu.CompilerParams()
if "needs_layout_passes" in pltpu.CompilerParams.__dataclass_fields__:
  _SC_PARAMS = dataclasses.replace(_SC_PARAMS, needs_layout_passes=False)
_SC_PARAMS_LINEAR = dataclasses.replace(_SC_PARAMS, use_tc_tiling_on_sc=False)


def _edge_accumulate(d_feat):
  @functools.partial(
      pl.kernel,
      out_type=jax.ShapeDtypeStruct((2, NP, d_feat), jnp.float32),
      mesh=_MESH,
      compiler_params=_SC_PARAMS if d_feat == 128 else _SC_PARAMS_LINEAR,
      scratch_types=[
          pltpu.VMEM((CH,), jnp.int32),
          pltpu.VMEM((CH,), jnp.int32),
          pltpu.VMEM((CH,), jnp.float32),
          pltpu.VMEM((CH, d_feat), jnp.float32),
          pltpu.VMEM_SHARED((NP, d_feat), jnp.float32),
      ],
  )
  def k(src_hbm, row_hbm, col_hbm, w_hbm, out_hbm, row_v, col_v, w_v,
        rows_v, acc_sh):
    cid = lax.axis_index("c")
    sid = lax.axis_index("s")
    wid = cid * 16 + sid

    @pl.loop(0, CH)
    def _(i):
      for d in range(d_feat // 16):
        rows_v[i, pl.ds(d * 16, 16)] = jnp.zeros((16,), jnp.float32)

    @pl.loop(0, CHUNKS_PER_TILE)
    def _(j):
      pltpu.sync_copy(rows_v, acc_sh.at[pl.ds(sid * ROWS_PER_TILE + j * CH, CH)])

    plsc.subcore_barrier()

    @pl.loop(0, EPW // CH)
    def _(k):
      base = wid * EPW + k * CH
      pltpu.sync_copy(row_hbm.at[pl.ds(base, CH)], row_v)
      pltpu.sync_copy(col_hbm.at[pl.ds(base, CH)], col_v)
      pltpu.sync_copy(w_hbm.at[pl.ds(base, CH)], w_v)
      pltpu.sync_copy(src_hbm.at[row_v], rows_v)

      @pl.loop(0, CH)
      def _(i):
        w = plsc.load_gather(w_v, [jnp.full((16,), i, jnp.int32)])
        for d in range(d_feat // 16):
          sl = (i, pl.ds(d * 16, 16))
          rows_v[sl] = rows_v[sl] * w

      pltpu.sync_copy(rows_v, acc_sh.at[col_v], add=True)

    plsc.subcore_barrier()

    @pl.loop(0, CHUNKS_PER_TILE)
    def _(j):
      start = sid * ROWS_PER_TILE + j * CH
      pltpu.sync_copy(acc_sh.at[pl.ds(start, CH)],
                      out_hbm.at[cid, pl.ds(start, CH)])

  return k


@functools.partial(
    pl.kernel,
    out_type=jax.ShapeDtypeStruct((2, NP), jnp.float32),
    mesh=_MESH,
    compiler_params=_SC_PARAMS,
    scratch_types=[
        pltpu.VMEM((CH,), jnp.int32),
        pltpu.VMEM((CH,), jnp.float32),
        pltpu.VMEM_SHARED((NP,), jnp.float32),
    ],
)
def _degree_kernel(col_hbm, w_hbm, out_hbm, col_v, w_v, deg_sh):
  cid = lax.axis_index("c")
  sid = lax.axis_index("s")
  wid = cid * 16 + sid

  @pl.loop(0, CH // 16)
  def _(g):
    w_v[pl.ds(g * 16, 16)] = jnp.zeros((16,), jnp.float32)

  @pl.loop(0, CHUNKS_PER_TILE)
  def _(j):
    pltpu.sync_copy(w_v, deg_sh.at[pl.ds(sid * ROWS_PER_TILE + j * CH, CH)])

  plsc.subcore_barrier()

  @pl.loop(0, EPW // CH)
  def _(k):
    base = wid * EPW + k * CH
    pltpu.sync_copy(col_hbm.at[pl.ds(base, CH)], col_v)
    pltpu.sync_copy(w_hbm.at[pl.ds(base, CH)], w_v)
    pltpu.sync_copy(w_v, deg_sh.at[col_v], add=True)

  plsc.subcore_barrier()

  @pl.loop(0, CHUNKS_PER_TILE)
  def _(j):
    start = sid * ROWS_PER_TILE + j * CH
    pltpu.sync_copy(deg_sh.at[pl.ds(start, CH)], out_hbm.at[cid, pl.ds(start, CH)])


@functools.partial(
    pl.kernel,
    out_type=[
        jax.ShapeDtypeStruct((2, NP), jnp.float32),
        jax.ShapeDtypeStruct((2, NP, 64), jnp.float32),
    ],
    mesh=_MESH,
    compiler_params=_SC_PARAMS_LINEAR,
    scratch_types=[
        pltpu.VMEM((CH,), jnp.int32),
        pltpu.VMEM((CH,), jnp.int32),
        pltpu.VMEM((CH,), jnp.float32),
        pltpu.VMEM((CH,), jnp.float32),
        pltpu.VMEM((CH,), jnp.float32),
        pltpu.VMEM((CH,), jnp.float32),
        pltpu.VMEM((CH,), jnp.float32),
        pltpu.VMEM((CH, 64), jnp.float32),
        pltpu.VMEM_SHARED((NP,), jnp.float32),
        pltpu.VMEM_SHARED((NP, 64), jnp.float32),
    ],
)
def _gat_edge_kernel(h3_hbm, asrc_hbm, adst_hbm, off_hbm, row_hbm, col_hbm,
                     valid_hbm, s_out, acc_out, row_v, col_v, val_v, as_v,
                     ad_v, off_v, ex_v, rows_v, s_sh, acc_sh):
  cid = lax.axis_index("c")
  sid = lax.axis_index("s")
  wid = cid * 16 + sid

  @pl.loop(0, CH)
  def _(i):
    for d in range(4):
      rows_v[i, pl.ds(d * 16, 16)] = jnp.zeros((16,), jnp.float32)

  @pl.loop(0, CH // 16)
  def _(g):
    ex_v[pl.ds(g * 16, 16)] = jnp.zeros((16,), jnp.float32)

  @pl.loop(0, CHUNKS_PER_TILE)
  def _(j):
    start = sid * ROWS_PER_TILE + j * CH
    pltpu.sync_copy(rows_v, acc_sh.at[pl.ds(start, CH)])
    pltpu.sync_copy(ex_v, s_sh.at[pl.ds(start, CH)])

  plsc.subcore_barrier()

  @pl.loop(0, EPW // CH)
  def _(k):
    base = wid * EPW + k * CH
    pltpu.sync_copy(row_hbm.at[pl.ds(base, CH)], row_v)
    pltpu.sync_copy(col_hbm.at[pl.ds(base, CH)], col_v)
    pltpu.sync_copy(valid_hbm.at[pl.ds(base, CH)], val_v)
    pltpu.sync_copy(asrc_hbm.at[row_v], as_v)
    pltpu.sync_copy(adst_hbm.at[col_v], ad_v)
    pltpu.sync_copy(off_hbm.at[col_v], off_v)
    pltpu.sync_copy(h3_hbm.at[row_v], rows_v)

    @pl.loop(0, CH // 16)
    def _(g):
      sl = pl.ds(g * 16, 16)
      z = as_v[sl] + ad_v[sl]
      e = jnp.where(z > 0.0, z, 0.2 * z)
      ex_v[sl] = jnp.exp(e - off_v[sl]) * val_v[sl]

    pltpu.sync_copy(ex_v, s_sh.at[col_v], add=True)

    @pl.loop(0, CH)
    def _(i):
      w = plsc.load_gather(ex_v, [jnp.full((16,), i, jnp.int32)])
      for d in range(4):
        sl = (i, pl.ds(d * 16, 16))
        rows_v[sl] = rows_v[sl] * w

    pltpu.sync_copy(rows_v, acc_sh.at[col_v], add=True)

  plsc.subcore_barrier()

  @pl.loop(0, CHUNKS_PER_TILE)
  def _(j):
    start = sid * ROWS_PER_TILE + j * CH
    pltpu.sync_copy(s_sh.at[pl.ds(start, CH)], s_out.at[cid, pl.ds(start, CH)])
    pltpu.sync_copy(acc_sh.at[pl.ds(start, CH)],
                    acc_out.at[cid, pl.ds(start, CH)])


BR = 2000
GRID = N_NODES // BR


def _rb(d):
  return pl.BlockSpec((BR, d), lambda i: (i, 0))


def _full(s0, s1):
  return pl.BlockSpec((s0, s1), lambda i: (0, 0))


def _row_call(body, in_specs, out_shape, out_specs):
  return pl.pallas_call(body, grid=(GRID,), in_specs=in_specs,
                        out_shape=out_shape, out_specs=out_specs)


_DOT = functools.partial(jnp.dot, preferred_element_type=jnp.float32,
                         precision=lax.Precision.HIGHEST)


def _mm_kernel(x_ref, w_ref, o_ref):
  o_ref[...] = _DOT(x_ref[...], w_ref[...])


def _scale_kernel(hp_ref, d0_ref, d1_ref, g_ref, dinv_ref):
  deg = d0_ref[...] + d1_ref[...] + 1.0
  dinv = lax.rsqrt(deg)
  dinv_ref[...] = dinv
  g_ref[...] = hp_ref[...] * dinv


def _combine_kernel(a0_ref, a1_ref, hp_ref, dinv_ref, b_ref, w_ref,
                    hnext_ref, gnext_ref):
  dinv = dinv_ref[...]
  out = dinv * (a0_ref[...] + a1_ref[...]) + dinv * dinv * hp_ref[...] \
      + b_ref[...]
  h = out * (1.0 / (1.0 + jnp.exp(-out)))
  hp = _DOT(h, w_ref[...])
  hnext_ref[...] = hp
  gnext_ref[...] = hp * dinv


def _gat_mm_kernel(a0_ref, a1_ref, hp_ref, dinv_ref, b_ref, w_ref,
                   atts_ref, attd_ref, h3_ref, asrc_ref, adst_ref):
  dinv = dinv_ref[...]
  out = dinv * (a0_ref[...] + a1_ref[...]) + dinv * dinv * hp_ref[...] \
      + b_ref[...]
  h2 = out * (1.0 / (1.0 + jnp.exp(-out)))
  h3 = _DOT(h2, w_ref[...])
  h3_ref[...] = h3
  asrc_ref[...] = _DOT(h3, atts_ref[...])
  adst_ref[...] = _DOT(h3, attd_ref[...])


def _att_prep_kernel(asrc_ref, adst_ref, off_ref, exs_ref):
  asrc = asrc_ref[...]
  adst = adst_ref[...]
  amax = jnp.max(asrc)
  es = asrc + adst
  e_self = jnp.where(es > 0.0, es, 0.2 * es)
  off = jnp.maximum(e_self, adst + amax)
  off_ref[...] = off
  exs_ref[...] = jnp.exp(e_self - off)


def _final_kernel(a0_ref, a1_ref, s0_ref, s1_ref, exs_ref, h3_ref, b_ref,
                  o_ref):
  s = s0_ref[...] + s1_ref[...] + exs_ref[...]
  num = a0_ref[...] + a1_ref[...] + exs_ref[...] * h3_ref[...]
  o3 = num / s + b_ref[...]
  m = jnp.max(o3, axis=1, keepdims=True)
  e = jnp.exp(o3 - m)
  o_ref[...] = e / jnp.sum(e, axis=1, keepdims=True)


def kernel(x, edge_index, edge_weight, W1, b1, W2, b2, W3, att_src, att_dst,
           b3):
  n = N_NODES
  row, col = edge_index[0], edge_index[1]

  pad = EPAD - N_EDGES
  pad_idx = (jnp.arange(pad, dtype=jnp.int32) * 8) % n
  row_p = jnp.concatenate([row, pad_idx])
  col_p = jnp.concatenate([col, pad_idx])
  ew_p = jnp.concatenate([edge_weight, jnp.zeros((pad,), jnp.float32)])
  valid_p = jnp.concatenate(
      [jnp.ones((N_EDGES,), jnp.float32), jnp.zeros((pad,), jnp.float32)])

  deg_parts = _degree_kernel(col_p, ew_p)
  h1p = _row_call(_mm_kernel, [_rb(128), _full(128, 128)],
                  jax.ShapeDtypeStruct((n, 128), jnp.float32),
                  _rb(128))(x, W1)

  d0 = deg_parts[0, :n].reshape(n, 1)
  d1 = deg_parts[1, :n].reshape(n, 1)
  g1, dinv = _row_call(
      _scale_kernel, [_rb(128), _rb(1), _rb(1)],
      (jax.ShapeDtypeStruct((n, 128), jnp.float32),
       jax.ShapeDtypeStruct((n, 1), jnp.float32)),
      (_rb(128), _rb(1)))(h1p, d0, d1)

  acc1 = _edge_accumulate(128)(g1, row_p, col_p, ew_p)
  h2p, g2 = _row_call(
      _combine_kernel,
      [_rb(128), _rb(128), _rb(128), _rb(1), _full(1, 128), _full(128, 64)],
      (jax.ShapeDtypeStruct((n, 64), jnp.float32),
       jax.ShapeDtypeStruct((n, 64), jnp.float32)),
      (_rb(64), _rb(64)))(
          acc1[0, :n], acc1[1, :n], h1p, dinv, b1.reshape(1, 128), W2)

  acc2 = _edge_accumulate(64)(g2, row_p, col_p, ew_p)
  h3, asrc, adst = _row_call(
      _gat_mm_kernel,
      [_rb(64), _rb(64), _rb(64), _rb(1), _full(1, 64), _full(64, 64),
       _full(64, 1), _full(64, 1)],
      (jax.ShapeDtypeStruct((n, 64), jnp.float32),
       jax.ShapeDtypeStruct((n, 1), jnp.float32),
       jax.ShapeDtypeStruct((n, 1), jnp.float32)),
      (_rb(64), _rb(1), _rb(1)))(
          acc2[0, :n], acc2[1, :n], h2p, dinv, b2.reshape(1, 64), W3,
          att_src.reshape(64, 1), att_dst.reshape(64, 1))

  off, exs = pl.pallas_call(
      _att_prep_kernel,
      out_shape=(jax.ShapeDtypeStruct((n, 1), jnp.float32),
                 jax.ShapeDtypeStruct((n, 1), jnp.float32)))(asrc, adst)

  s_parts, acc3 = _gat_edge_kernel(
      h3, asrc.reshape(n), adst.reshape(n), off.reshape(n), row_p, col_p,
      valid_p)

  out = _row_call(
      _final_kernel,
      [_rb(64), _rb(64), _rb(1), _rb(1), _rb(1), _rb(64), _full(1, 64)],
      jax.ShapeDtypeStruct((n, 64), jnp.float32),
      _rb(64))(
          acc3[0, :n], acc3[1, :n], s_parts[0, :n].reshape(n, 1),
          s_parts[1, :n].reshape(n, 1), exs, h3, b3.reshape(1, 64))
  return out


# speedup vs baseline: 17.1670x; 1.1745x over previous
import dataclasses
import functools

import jax
import jax.numpy as jnp
from jax import lax
from jax.experimental import pallas as pl
from jax.experimental.pallas import tpu as pltpu
from jax.experimental.pallas import tpu_sc as plsc

N_NODES = 10000
N_EDGES = 320000
NP = 10240
N_WORKERS = 32
CH = 128
EPW = 10240
EPAD = EPW * N_WORKERS
ROWS_PER_TILE = NP // 16
CHUNKS_PER_TILE = ROWS_PER_TILE // CH

_MESH = plsc.VectorSubcoreMesh(core_axis_name="c", subcore_axis_name="s")

_SC_PARAMS = pltpu.CompilerParams()
if "needs_layout_passes" in pltpu.CompilerParams.__dataclass_fields__:
  _SC_PARAMS = dataclasses.replace(_SC_PARAMS, needs_layout_passes=False)
_SC_PARAMS_LINEAR = dataclasses.replace(_SC_PARAMS, use_tc_tiling_on_sc=False)


def _edge_accumulate(d_feat, sub):
  CHUNK = CH * sub
  NCH = EPW // CHUNK

  @functools.partial(
      pl.kernel,
      out_type=jax.ShapeDtypeStruct((2, NP, d_feat), jnp.float32),
      mesh=_MESH,
      compiler_params=_SC_PARAMS if d_feat == 128 else _SC_PARAMS_LINEAR,
      scratch_types=[
          pltpu.VMEM((2, sub, CH), jnp.int32),
          pltpu.VMEM((2, sub, CH), jnp.int32),
          pltpu.VMEM((2, sub, CH), jnp.float32),
          pltpu.VMEM((CHUNK, d_feat), jnp.float32),
          pltpu.VMEM_SHARED((NP, d_feat), jnp.float32),
          pltpu.SemaphoreType.DMA,
          pltpu.SemaphoreType.DMA,
      ],
  )
  def k(src_hbm, row_hbm, col_hbm, w_hbm, out_hbm, row_v, col_v, w_v,
        rows_v, acc_sh, si0, si1):
    cid = lax.axis_index("c")
    sid = lax.axis_index("s")
    wid = cid * 16 + sid
    si = (si0, si1)

    def start_idx(chunk, b):
      base = wid * EPW + chunk * CHUNK
      for s in range(sub):
        pltpu.async_copy(row_hbm.at[pl.ds(base + s * CH, CH)],
                         row_v.at[b, s], si[b])
        pltpu.async_copy(col_hbm.at[pl.ds(base + s * CH, CH)],
                         col_v.at[b, s], si[b])
        pltpu.async_copy(w_hbm.at[pl.ds(base + s * CH, CH)],
                         w_v.at[b, s], si[b])

    def wait_idx(b):
      for s in range(sub):
        pltpu.make_async_copy(row_hbm.at[pl.ds(0, CH)], row_v.at[b, s],
                              si[b]).wait()
        pltpu.make_async_copy(col_hbm.at[pl.ds(0, CH)], col_v.at[b, s],
                              si[b]).wait()
        pltpu.make_async_copy(w_hbm.at[pl.ds(0, CH)], w_v.at[b, s],
                              si[b]).wait()

    @pl.loop(0, CH)
    def _(i):
      for d in range(d_feat // 16):
        rows_v[i, pl.ds(d * 16, 16)] = jnp.zeros((16,), jnp.float32)

    @pl.loop(0, CHUNKS_PER_TILE)
    def _(j):
      pltpu.sync_copy(rows_v.at[pl.ds(0, CH)],
                      acc_sh.at[pl.ds(sid * ROWS_PER_TILE + j * CH, CH)])

    plsc.subcore_barrier()

    start_idx(0, 0)
    start_idx(1, 1)

    @pl.loop(0, NCH // 2)
    def _(j):
      for b in (0, 1):
        k = 2 * j + b
        wait_idx(b)

        for s in range(sub):
          pltpu.sync_copy(src_hbm.at[row_v.at[b, s]],
                          rows_v.at[pl.ds(s * CH, CH)])

        for s in range(sub):
          @pl.loop(0, CH)
          def _(i):
            w = plsc.load_gather(w_v.at[b, s], [jnp.full((16,), i, jnp.int32)])
            for d in range(d_feat // 16):
              sl = (s * CH + i, pl.ds(d * 16, 16))
              rows_v[sl] = rows_v[sl] * w

        for s in range(sub):
          pltpu.sync_copy(rows_v.at[pl.ds(s * CH, CH)],
                          acc_sh.at[col_v.at[b, s]], add=True)

        nk = jnp.where(k + 2 >= NCH, k + 2 - NCH, k + 2)
        start_idx(nk, b)

    wait_idx(0)
    wait_idx(1)

    plsc.subcore_barrier()

    @pl.loop(0, CHUNKS_PER_TILE)
    def _(j):
      start = sid * ROWS_PER_TILE + j * CH
      pltpu.sync_copy(acc_sh.at[pl.ds(start, CH)],
                      out_hbm.at[cid, pl.ds(start, CH)])

  return k


@functools.partial(
    pl.kernel,
    out_type=jax.ShapeDtypeStruct((2, NP), jnp.float32),
    mesh=_MESH,
    compiler_params=_SC_PARAMS,
    scratch_types=[
        pltpu.VMEM((CH,), jnp.int32),
        pltpu.VMEM((CH,), jnp.float32),
        pltpu.VMEM_SHARED((NP,), jnp.float32),
    ],
)
def _degree_kernel(col_hbm, w_hbm, out_hbm, col_v, w_v, deg_sh):
  cid = lax.axis_index("c")
  sid = lax.axis_index("s")
  wid = cid * 16 + sid

  @pl.loop(0, CH // 16)
  def _(g):
    w_v[pl.ds(g * 16, 16)] = jnp.zeros((16,), jnp.float32)

  @pl.loop(0, CHUNKS_PER_TILE)
  def _(j):
    pltpu.sync_copy(w_v, deg_sh.at[pl.ds(sid * ROWS_PER_TILE + j * CH, CH)])

  plsc.subcore_barrier()

  @pl.loop(0, EPW // CH)
  def _(k):
    base = wid * EPW + k * CH
    pltpu.sync_copy(col_hbm.at[pl.ds(base, CH)], col_v)
    pltpu.sync_copy(w_hbm.at[pl.ds(base, CH)], w_v)
    pltpu.sync_copy(w_v, deg_sh.at[col_v], add=True)

  plsc.subcore_barrier()

  @pl.loop(0, CHUNKS_PER_TILE)
  def _(j):
    start = sid * ROWS_PER_TILE + j * CH
    pltpu.sync_copy(deg_sh.at[pl.ds(start, CH)], out_hbm.at[cid, pl.ds(start, CH)])


@functools.partial(
    pl.kernel,
    out_type=[
        jax.ShapeDtypeStruct((2, NP), jnp.float32),
        jax.ShapeDtypeStruct((2, NP, 64), jnp.float32),
    ],
    mesh=_MESH,
    compiler_params=_SC_PARAMS_LINEAR,
    scratch_types=[
        pltpu.VMEM((CH,), jnp.int32),
        pltpu.VMEM((CH,), jnp.int32),
        pltpu.VMEM((CH,), jnp.float32),
        pltpu.VMEM((CH,), jnp.float32),
        pltpu.VMEM((CH,), jnp.float32),
        pltpu.VMEM((CH,), jnp.float32),
        pltpu.VMEM((CH,), jnp.float32),
        pltpu.VMEM((CH, 64), jnp.float32),
        pltpu.VMEM_SHARED((NP,), jnp.float32),
        pltpu.VMEM_SHARED((NP, 64), jnp.float32),
    ],
)
def _gat_edge_kernel(h3_hbm, asrc_hbm, adst_hbm, off_hbm, row_hbm, col_hbm,
                     valid_hbm, s_out, acc_out, row_v, col_v, val_v, as_v,
                     ad_v, off_v, ex_v, rows_v, s_sh, acc_sh):
  cid = lax.axis_index("c")
  sid = lax.axis_index("s")
  wid = cid * 16 + sid

  @pl.loop(0, CH)
  def _(i):
    for d in range(4):
      rows_v[i, pl.ds(d * 16, 16)] = jnp.zeros((16,), jnp.float32)

  @pl.loop(0, CH // 16)
  def _(g):
    ex_v[pl.ds(g * 16, 16)] = jnp.zeros((16,), jnp.float32)

  @pl.loop(0, CHUNKS_PER_TILE)
  def _(j):
    start = sid * ROWS_PER_TILE + j * CH
    pltpu.sync_copy(rows_v, acc_sh.at[pl.ds(start, CH)])
    pltpu.sync_copy(ex_v, s_sh.at[pl.ds(start, CH)])

  plsc.subcore_barrier()

  @pl.loop(0, EPW // CH)
  def _(k):
    base = wid * EPW + k * CH
    pltpu.sync_copy(row_hbm.at[pl.ds(base, CH)], row_v)
    pltpu.sync_copy(col_hbm.at[pl.ds(base, CH)], col_v)
    pltpu.sync_copy(valid_hbm.at[pl.ds(base, CH)], val_v)
    pltpu.sync_copy(asrc_hbm.at[row_v], as_v)
    pltpu.sync_copy(adst_hbm.at[col_v], ad_v)
    pltpu.sync_copy(off_hbm.at[col_v], off_v)
    pltpu.sync_copy(h3_hbm.at[row_v], rows_v)

    @pl.loop(0, CH // 16)
    def _(g):
      sl = pl.ds(g * 16, 16)
      z = as_v[sl] + ad_v[sl]
      e = jnp.where(z > 0.0, z, 0.2 * z)
      ex_v[sl] = jnp.exp(e - off_v[sl]) * val_v[sl]

    pltpu.sync_copy(ex_v, s_sh.at[col_v], add=True)

    @pl.loop(0, CH)
    def _(i):
      w = plsc.load_gather(ex_v, [jnp.full((16,), i, jnp.int32)])
      for d in range(4):
        sl = (i, pl.ds(d * 16, 16))
        rows_v[sl] = rows_v[sl] * w

    pltpu.sync_copy(rows_v, acc_sh.at[col_v], add=True)

  plsc.subcore_barrier()

  @pl.loop(0, CHUNKS_PER_TILE)
  def _(j):
    start = sid * ROWS_PER_TILE + j * CH
    pltpu.sync_copy(s_sh.at[pl.ds(start, CH)], s_out.at[cid, pl.ds(start, CH)])
    pltpu.sync_copy(acc_sh.at[pl.ds(start, CH)],
                    acc_out.at[cid, pl.ds(start, CH)])


BR = 2000
GRID = N_NODES // BR


def _rb(d):
  return pl.BlockSpec((BR, d), lambda i: (i, 0))


def _full(s0, s1):
  return pl.BlockSpec((s0, s1), lambda i: (0, 0))


def _row_call(body, in_specs, out_shape, out_specs):
  return pl.pallas_call(body, grid=(GRID,), in_specs=in_specs,
                        out_shape=out_shape, out_specs=out_specs)


_DOT = functools.partial(jnp.dot, preferred_element_type=jnp.float32,
                         precision=lax.Precision.HIGHEST)


def _mm_kernel(x_ref, w_ref, o_ref):
  o_ref[...] = _DOT(x_ref[...], w_ref[...])


def _scale_kernel(hp_ref, d0_ref, d1_ref, g_ref, dinv_ref):
  deg = d0_ref[...] + d1_ref[...] + 1.0
  dinv = lax.rsqrt(deg)
  dinv_ref[...] = dinv
  g_ref[...] = hp_ref[...] * dinv


def _combine_kernel(a0_ref, a1_ref, hp_ref, dinv_ref, b_ref, w_ref,
                    hnext_ref, gnext_ref):
  dinv = dinv_ref[...]
  out = dinv * (a0_ref[...] + a1_ref[...]) + dinv * dinv * hp_ref[...] \
      + b_ref[...]
  h = out * (1.0 / (1.0 + jnp.exp(-out)))
  hp = _DOT(h, w_ref[...])
  hnext_ref[...] = hp
  gnext_ref[...] = hp * dinv


def _gat_mm_kernel(a0_ref, a1_ref, hp_ref, dinv_ref, b_ref, w_ref,
                   atts_ref, attd_ref, h3_ref, asrc_ref, adst_ref):
  dinv = dinv_ref[...]
  out = dinv * (a0_ref[...] + a1_ref[...]) + dinv * dinv * hp_ref[...] \
      + b_ref[...]
  h2 = out * (1.0 / (1.0 + jnp.exp(-out)))
  h3 = _DOT(h2, w_ref[...])
  h3_ref[...] = h3
  asrc_ref[...] = _DOT(h3, atts_ref[...])
  adst_ref[...] = _DOT(h3, attd_ref[...])


def _att_prep_kernel(asrc_ref, adst_ref, off_ref, exs_ref):
  asrc = asrc_ref[...]
  adst = adst_ref[...]
  amax = jnp.max(asrc)
  es = asrc + adst
  e_self = jnp.where(es > 0.0, es, 0.2 * es)
  off = jnp.maximum(e_self, adst + amax)
  off_ref[...] = off
  exs_ref[...] = jnp.exp(e_self - off)


def _final_kernel(a0_ref, a1_ref, s0_ref, s1_ref, exs_ref, h3_ref, b_ref,
                  o_ref):
  s = s0_ref[...] + s1_ref[...] + exs_ref[...]
  num = a0_ref[...] + a1_ref[...] + exs_ref[...] * h3_ref[...]
  o3 = num / s + b_ref[...]
  m = jnp.max(o3, axis=1, keepdims=True)
  e = jnp.exp(o3 - m)
  o_ref[...] = e / jnp.sum(e, axis=1, keepdims=True)


def kernel(x, edge_index, edge_weight, W1, b1, W2, b2, W3, att_src, att_dst,
           b3):
  n = N_NODES
  row, col = edge_index[0], edge_index[1]

  pad = EPAD - N_EDGES
  pad_idx = (jnp.arange(pad, dtype=jnp.int32) * 8) % n
  row_p = jnp.concatenate([row, pad_idx])
  col_p = jnp.concatenate([col, pad_idx])
  ew_p = jnp.concatenate([edge_weight, jnp.zeros((pad,), jnp.float32)])
  valid_p = jnp.concatenate(
      [jnp.ones((N_EDGES,), jnp.float32), jnp.zeros((pad,), jnp.float32)])

  deg_parts = _degree_kernel(col_p, ew_p)
  h1p = _row_call(_mm_kernel, [_rb(128), _full(128, 128)],
                  jax.ShapeDtypeStruct((n, 128), jnp.float32),
                  _rb(128))(x, W1)

  d0 = deg_parts[0, :n].reshape(n, 1)
  d1 = deg_parts[1, :n].reshape(n, 1)
  g1, dinv = _row_call(
      _scale_kernel, [_rb(128), _rb(1), _rb(1)],
      (jax.ShapeDtypeStruct((n, 128), jnp.float32),
       jax.ShapeDtypeStruct((n, 1), jnp.float32)),
      (_rb(128), _rb(1)))(h1p, d0, d1)

  acc1 = _edge_accumulate(128, 1)(g1, row_p, col_p, ew_p)
  h2p, g2 = _row_call(
      _combine_kernel,
      [_rb(128), _rb(128), _rb(128), _rb(1), _full(1, 128), _full(128, 64)],
      (jax.ShapeDtypeStruct((n, 64), jnp.float32),
       jax.ShapeDtypeStruct((n, 64), jnp.float32)),
      (_rb(64), _rb(64)))(
          acc1[0, :n], acc1[1, :n], h1p, dinv, b1.reshape(1, 128), W2)

  acc2 = _edge_accumulate(64, 2)(g2, row_p, col_p, ew_p)
  h3, asrc, adst = _row_call(
      _gat_mm_kernel,
      [_rb(64), _rb(64), _rb(64), _rb(1), _full(1, 64), _full(64, 64),
       _full(64, 1), _full(64, 1)],
      (jax.ShapeDtypeStruct((n, 64), jnp.float32),
       jax.ShapeDtypeStruct((n, 1), jnp.float32),
       jax.ShapeDtypeStruct((n, 1), jnp.float32)),
      (_rb(64), _rb(1), _rb(1)))(
          acc2[0, :n], acc2[1, :n], h2p, dinv, b2.reshape(1, 64), W3,
          att_src.reshape(64, 1), att_dst.reshape(64, 1))

  off, exs = pl.pallas_call(
      _att_prep_kernel,
      out_shape=(jax.ShapeDtypeStruct((n, 1), jnp.float32),
                 jax.ShapeDtypeStruct((n, 1), jnp.float32)))(asrc, adst)

  s_parts, acc3 = _gat_edge_kernel(
      h3, asrc.reshape(n), adst.reshape(n), off.reshape(n), row_p, col_p,
      valid_p)

  out = _row_call(
      _final_kernel,
      [_rb(64), _rb(64), _rb(1), _rb(1), _rb(1), _rb(64), _full(1, 64)],
      jax.ShapeDtypeStruct((n, 64), jnp.float32),
      _rb(64))(
          acc3[0, :n], acc3[1, :n], s_parts[0, :n].reshape(n, 1),
          s_parts[1, :n].reshape(n, 1), exs, h3, b3.reshape(1, 64))
  return out


# speedup vs baseline: 20.9050x; 1.2177x over previous
import dataclasses
import functools

import jax
import jax.numpy as jnp
from jax import lax
from jax.experimental import pallas as pl
from jax.experimental.pallas import tpu as pltpu
from jax.experimental.pallas import tpu_sc as plsc

N_NODES = 10000
N_EDGES = 320000
NP = 10240
N_WORKERS = 32
CH = 128
EPW = 10240
EPAD = EPW * N_WORKERS
ROWS_PER_TILE = NP // 16
CHUNKS_PER_TILE = ROWS_PER_TILE // CH

_MESH = plsc.VectorSubcoreMesh(core_axis_name="c", subcore_axis_name="s")

_SC_PARAMS = pltpu.CompilerParams()
if "needs_layout_passes" in pltpu.CompilerParams.__dataclass_fields__:
  _SC_PARAMS = dataclasses.replace(_SC_PARAMS, needs_layout_passes=False)
_SC_PARAMS_LINEAR = dataclasses.replace(_SC_PARAMS, use_tc_tiling_on_sc=False)


def _edge_accumulate(d_feat, sub):
  CHUNK = CH * sub
  NCH = EPW // CHUNK

  @functools.partial(
      pl.kernel,
      out_type=jax.ShapeDtypeStruct((2, NP, d_feat), jnp.float32),
      mesh=_MESH,
      compiler_params=_SC_PARAMS if d_feat == 128 else _SC_PARAMS_LINEAR,
      scratch_types=[
          pltpu.VMEM((2, sub, CH), jnp.int32),
          pltpu.VMEM((2, sub, CH), jnp.int32),
          pltpu.VMEM((2, sub, CH), jnp.float32),
          pltpu.VMEM((CHUNK, d_feat), jnp.float32),
          pltpu.VMEM_SHARED((NP, d_feat), jnp.float32),
          pltpu.SemaphoreType.DMA,
          pltpu.SemaphoreType.DMA,
      ],
  )
  def k(src_hbm, row_hbm, col_hbm, w_hbm, out_hbm, row_v, col_v, w_v,
        rows_v, acc_sh, si0, si1):
    cid = lax.axis_index("c")
    sid = lax.axis_index("s")
    wid = cid * 16 + sid
    si = (si0, si1)

    def start_idx(chunk, b):
      base = wid * EPW + chunk * CHUNK
      for s in range(sub):
        pltpu.async_copy(row_hbm.at[pl.ds(base + s * CH, CH)],
                         row_v.at[b, s], si[b])
        pltpu.async_copy(col_hbm.at[pl.ds(base + s * CH, CH)],
                         col_v.at[b, s], si[b])
        pltpu.async_copy(w_hbm.at[pl.ds(base + s * CH, CH)],
                         w_v.at[b, s], si[b])

    def wait_idx(b):
      for s in range(sub):
        pltpu.make_async_copy(row_hbm.at[pl.ds(0, CH)], row_v.at[b, s],
                              si[b]).wait()
        pltpu.make_async_copy(col_hbm.at[pl.ds(0, CH)], col_v.at[b, s],
                              si[b]).wait()
        pltpu.make_async_copy(w_hbm.at[pl.ds(0, CH)], w_v.at[b, s],
                              si[b]).wait()

    @pl.loop(0, CH)
    def _(i):
      for d in range(d_feat // 16):
        rows_v[i, pl.ds(d * 16, 16)] = jnp.zeros((16,), jnp.float32)

    @pl.loop(0, CHUNKS_PER_TILE)
    def _(j):
      pltpu.sync_copy(rows_v.at[pl.ds(0, CH)],
                      acc_sh.at[pl.ds(sid * ROWS_PER_TILE + j * CH, CH)])

    plsc.subcore_barrier()

    start_idx(0, 0)
    start_idx(1, 1)

    @pl.loop(0, NCH // 2)
    def _(j):
      for b in (0, 1):
        k = 2 * j + b
        wait_idx(b)

        for s in range(sub):
          pltpu.sync_copy(src_hbm.at[row_v.at[b, s]],
                          rows_v.at[pl.ds(s * CH, CH)])

        for s in range(sub):
          @pl.loop(0, CH)
          def _(i):
            w = plsc.load_gather(w_v.at[b, s], [jnp.full((16,), i, jnp.int32)])
            for d in range(d_feat // 16):
              sl = (s * CH + i, pl.ds(d * 16, 16))
              rows_v[sl] = rows_v[sl] * w

        for s in range(sub):
          pltpu.sync_copy(rows_v.at[pl.ds(s * CH, CH)],
                          acc_sh.at[col_v.at[b, s]], add=True)

        nk = jnp.where(k + 2 >= NCH, k + 2 - NCH, k + 2)
        start_idx(nk, b)

    wait_idx(0)
    wait_idx(1)

    plsc.subcore_barrier()

    @pl.loop(0, CHUNKS_PER_TILE)
    def _(j):
      start = sid * ROWS_PER_TILE + j * CH
      pltpu.sync_copy(acc_sh.at[pl.ds(start, CH)],
                      out_hbm.at[cid, pl.ds(start, CH)])

  return k


_DEG_SUB = 4


@functools.partial(
    pl.kernel,
    out_type=jax.ShapeDtypeStruct((2, NP), jnp.float32),
    mesh=_MESH,
    compiler_params=_SC_PARAMS,
    scratch_types=[
        pltpu.VMEM((2, _DEG_SUB, CH), jnp.int32),
        pltpu.VMEM((2, _DEG_SUB, CH), jnp.float32),
        pltpu.VMEM_SHARED((NP,), jnp.float32),
        pltpu.SemaphoreType.DMA,
        pltpu.SemaphoreType.DMA,
    ],
)
def _degree_kernel(col_hbm, w_hbm, out_hbm, col_v, w_v, deg_sh, si0, si1):
  cid = lax.axis_index("c")
  sid = lax.axis_index("s")
  wid = cid * 16 + sid
  si = (si0, si1)
  chunk = _DEG_SUB * CH
  nch = EPW // chunk

  def start_idx(k, b):
    base = wid * EPW + k * chunk
    for s in range(_DEG_SUB):
      pltpu.async_copy(col_hbm.at[pl.ds(base + s * CH, CH)], col_v.at[b, s],
                       si[b])
      pltpu.async_copy(w_hbm.at[pl.ds(base + s * CH, CH)], w_v.at[b, s],
                       si[b])

  def wait_idx(b):
    for s in range(_DEG_SUB):
      pltpu.make_async_copy(col_hbm.at[pl.ds(0, CH)], col_v.at[b, s],
                            si[b]).wait()
      pltpu.make_async_copy(w_hbm.at[pl.ds(0, CH)], w_v.at[b, s],
                            si[b]).wait()

  @pl.loop(0, CH // 16)
  def _(g):
    w_v[0, 0, pl.ds(g * 16, 16)] = jnp.zeros((16,), jnp.float32)

  @pl.loop(0, CHUNKS_PER_TILE)
  def _(j):
    pltpu.sync_copy(w_v.at[0, 0],
                    deg_sh.at[pl.ds(sid * ROWS_PER_TILE + j * CH, CH)])

  plsc.subcore_barrier()
  start_idx(0, 0)
  start_idx(1, 1)

  @pl.loop(0, nch // 2)
  def _(j):
    for b in (0, 1):
      k = 2 * j + b
      wait_idx(b)
      for s in range(_DEG_SUB):
        pltpu.sync_copy(w_v.at[b, s], deg_sh.at[col_v.at[b, s]], add=True)
      nk = jnp.where(k + 2 >= nch, k + 2 - nch, k + 2)
      start_idx(nk, b)

  wait_idx(0)
  wait_idx(1)
  plsc.subcore_barrier()

  @pl.loop(0, CHUNKS_PER_TILE)
  def _(j):
    start = sid * ROWS_PER_TILE + j * CH
    pltpu.sync_copy(deg_sh.at[pl.ds(start, CH)], out_hbm.at[cid, pl.ds(start, CH)])


_GAT_SUB = 2
_GAT_CHUNK = _GAT_SUB * CH


@functools.partial(
    pl.kernel,
    out_type=[
        jax.ShapeDtypeStruct((2, NP), jnp.float32),
        jax.ShapeDtypeStruct((2, NP, 64), jnp.float32),
    ],
    mesh=_MESH,
    compiler_params=_SC_PARAMS_LINEAR,
    scratch_types=[
        pltpu.VMEM((2, _GAT_SUB, CH), jnp.int32),
        pltpu.VMEM((2, _GAT_SUB, CH), jnp.int32),
        pltpu.VMEM((_GAT_CHUNK,), jnp.float32),
        pltpu.VMEM((_GAT_CHUNK, 2), jnp.float32),
        pltpu.VMEM((_GAT_CHUNK,), jnp.float32),
        pltpu.VMEM((_GAT_CHUNK, 64), jnp.float32),
        pltpu.VMEM_SHARED((NP,), jnp.float32),
        pltpu.VMEM_SHARED((NP, 64), jnp.float32),
        pltpu.SemaphoreType.DMA,
        pltpu.SemaphoreType.DMA,
    ],
)
def _gat_edge_kernel(h3_hbm, asrc_hbm, pq_hbm, row_hbm, col_hbm,
                     s_out, acc_out, row_v, col_v, as_v, pq_v, ex_v, rows_v,
                     s_sh, acc_sh, si0, si1):
  cid = lax.axis_index("c")
  sid = lax.axis_index("s")
  wid = cid * 16 + sid
  si = (si0, si1)
  nch = EPW // _GAT_CHUNK

  def start_idx(k, b):
    base = wid * EPW + k * _GAT_CHUNK
    for s in range(_GAT_SUB):
      pltpu.async_copy(row_hbm.at[pl.ds(base + s * CH, CH)], row_v.at[b, s],
                       si[b])
      pltpu.async_copy(col_hbm.at[pl.ds(base + s * CH, CH)], col_v.at[b, s],
                       si[b])

  def wait_idx(b):
    for s in range(_GAT_SUB):
      pltpu.make_async_copy(row_hbm.at[pl.ds(0, CH)], row_v.at[b, s],
                            si[b]).wait()
      pltpu.make_async_copy(col_hbm.at[pl.ds(0, CH)], col_v.at[b, s],
                            si[b]).wait()

  @pl.loop(0, CH)
  def _(i):
    for d in range(4):
      rows_v[i, pl.ds(d * 16, 16)] = jnp.zeros((16,), jnp.float32)

  @pl.loop(0, CH // 16)
  def _(g):
    ex_v[pl.ds(g * 16, 16)] = jnp.zeros((16,), jnp.float32)

  @pl.loop(0, CHUNKS_PER_TILE)
  def _(j):
    start = sid * ROWS_PER_TILE + j * CH
    pltpu.sync_copy(rows_v.at[pl.ds(0, CH)], acc_sh.at[pl.ds(start, CH)])
    pltpu.sync_copy(ex_v.at[pl.ds(0, CH)], s_sh.at[pl.ds(start, CH)])

  plsc.subcore_barrier()
  start_idx(0, 0)
  start_idx(1, 1)

  @pl.loop(0, nch // 2)
  def _(j):
    for b in (0, 1):
      k = 2 * j + b
      wait_idx(b)

      for s in range(_GAT_SUB):
        pltpu.sync_copy(asrc_hbm.at[row_v.at[b, s]],
                        as_v.at[pl.ds(s * CH, CH)])
        pltpu.sync_copy(pq_hbm.at[col_v.at[b, s]],
                        pq_v.at[pl.ds(s * CH, CH)])
        pltpu.sync_copy(h3_hbm.at[row_v.at[b, s]],
                        rows_v.at[pl.ds(s * CH, CH)])

      @pl.loop(0, _GAT_CHUNK // 16)
      def _(g):
        lane = lax.iota(jnp.int32, 16) + g * 16
        ad = plsc.load_gather(pq_v, [lane, jnp.zeros((16,), jnp.int32)])
        off = plsc.load_gather(pq_v, [lane, jnp.ones((16,), jnp.int32)])
        sl = pl.ds(g * 16, 16)
        z = as_v[sl] + ad
        e = jnp.where(z > 0.0, z, 0.2 * z)
        ex_v[sl] = jnp.exp(e - off)

      @pl.loop(0, _GAT_CHUNK)
      def _(i):
        w = plsc.load_gather(ex_v, [jnp.full((16,), i, jnp.int32)])
        for d in range(4):
          sl = (i, pl.ds(d * 16, 16))
          rows_v[sl] = rows_v[sl] * w

      for s in range(_GAT_SUB):
        pltpu.sync_copy(ex_v.at[pl.ds(s * CH, CH)],
                        s_sh.at[col_v.at[b, s]], add=True)
        pltpu.sync_copy(rows_v.at[pl.ds(s * CH, CH)],
                        acc_sh.at[col_v.at[b, s]], add=True)

      nk = jnp.where(k + 2 >= nch, k + 2 - nch, k + 2)
      start_idx(nk, b)

  wait_idx(0)
  wait_idx(1)
  plsc.subcore_barrier()

  @pl.loop(0, CHUNKS_PER_TILE)
  def _(j):
    start = sid * ROWS_PER_TILE + j * CH
    pltpu.sync_copy(s_sh.at[pl.ds(start, CH)], s_out.at[cid, pl.ds(start, CH)])
    pltpu.sync_copy(acc_sh.at[pl.ds(start, CH)],
                    acc_out.at[cid, pl.ds(start, CH)])


BR = 2000
GRID = N_NODES // BR


def _rb(d):
  return pl.BlockSpec((BR, d), lambda i: (i, 0))


def _full(s0, s1):
  return pl.BlockSpec((s0, s1), lambda i: (0, 0))


def _row_call(body, in_specs, out_shape, out_specs):
  return pl.pallas_call(body, grid=(GRID,), in_specs=in_specs,
                        out_shape=out_shape, out_specs=out_specs)


_DOT = functools.partial(jnp.dot, preferred_element_type=jnp.float32,
                         precision=lax.Precision.HIGHEST)


def _mm_kernel(x_ref, w_ref, o_ref):
  o_ref[...] = _DOT(x_ref[...], w_ref[...])


def _scale_kernel(hp_ref, d0_ref, d1_ref, g_ref, dinv_ref):
  deg = d0_ref[...] + d1_ref[...] + 1.0
  dinv = lax.rsqrt(deg)
  dinv_ref[...] = dinv
  g_ref[...] = hp_ref[...] * dinv


def _combine_kernel(a0_ref, a1_ref, hp_ref, dinv_ref, b_ref, w_ref,
                    hnext_ref, gnext_ref):
  dinv = dinv_ref[...]
  out = dinv * (a0_ref[...] + a1_ref[...]) + dinv * dinv * hp_ref[...] \
      + b_ref[...]
  h = out * (1.0 / (1.0 + jnp.exp(-out)))
  hp = _DOT(h, w_ref[...])
  hnext_ref[...] = hp
  gnext_ref[...] = hp * dinv


def _gat_mm_kernel(a0_ref, a1_ref, hp_ref, dinv_ref, b_ref, w_ref,
                   atts_ref, attd_ref, h3_ref, asrc_ref, adst_ref):
  dinv = dinv_ref[...]
  out = dinv * (a0_ref[...] + a1_ref[...]) + dinv * dinv * hp_ref[...] \
      + b_ref[...]
  h2 = out * (1.0 / (1.0 + jnp.exp(-out)))
  h3 = _DOT(h2, w_ref[...])
  h3_ref[...] = h3
  asrc_ref[...] = _DOT(h3, atts_ref[...])
  adst_ref[...] = _DOT(h3, attd_ref[...])


def _att_prep_kernel(asrc_ref, adst_ref, pq_ref, exs_ref):
  asrc = asrc_ref[...]
  adst = adst_ref[...]
  amax = jnp.max(asrc)
  es = asrc + adst
  e_self = jnp.where(es > 0.0, es, 0.2 * es)
  off = jnp.maximum(e_self, adst + amax)
  pq_ref[...] = jnp.concatenate([adst, off], axis=1)
  exs_ref[...] = jnp.exp(e_self - off)


def _final_kernel(a0_ref, a1_ref, s0_ref, s1_ref, exs_ref, h3_ref, b_ref,
                  o_ref):
  s = s0_ref[...] + s1_ref[...] + exs_ref[...]
  num = a0_ref[...] + a1_ref[...] + exs_ref[...] * h3_ref[...]
  o3 = num / s + b_ref[...]
  m = jnp.max(o3, axis=1, keepdims=True)
  e = jnp.exp(o3 - m)
  o_ref[...] = e / jnp.sum(e, axis=1, keepdims=True)


def kernel(x, edge_index, edge_weight, W1, b1, W2, b2, W3, att_src, att_dst,
           b3):
  n = N_NODES
  row, col = edge_index[0], edge_index[1]

  pad = EPAD - N_EDGES
  pad_idx = (jnp.arange(pad, dtype=jnp.int32) * 8) % n
  row_p = jnp.concatenate([row, pad_idx])
  col_p = jnp.concatenate([col, pad_idx])
  ew_p = jnp.concatenate([edge_weight, jnp.zeros((pad,), jnp.float32)])
  row_q = jnp.concatenate(
      [row, n + (jnp.arange(pad, dtype=jnp.int32) % 16)])

  deg_parts = _degree_kernel(col_p, ew_p)
  h1p = _row_call(_mm_kernel, [_rb(128), _full(128, 128)],
                  jax.ShapeDtypeStruct((n, 128), jnp.float32),
                  _rb(128))(x, W1)

  d0 = deg_parts[0, :n].reshape(n, 1)
  d1 = deg_parts[1, :n].reshape(n, 1)
  g1, dinv = _row_call(
      _scale_kernel, [_rb(128), _rb(1), _rb(1)],
      (jax.ShapeDtypeStruct((n, 128), jnp.float32),
       jax.ShapeDtypeStruct((n, 1), jnp.float32)),
      (_rb(128), _rb(1)))(h1p, d0, d1)

  acc1 = _edge_accumulate(128, 2)(g1, row_p, col_p, ew_p)
  h2p, g2 = _row_call(
      _combine_kernel,
      [_rb(128), _rb(128), _rb(128), _rb(1), _full(1, 128), _full(128, 64)],
      (jax.ShapeDtypeStruct((n, 64), jnp.float32),
       jax.ShapeDtypeStruct((n, 64), jnp.float32)),
      (_rb(64), _rb(64)))(
          acc1[0, :n], acc1[1, :n], h1p, dinv, b1.reshape(1, 128), W2)

  acc2 = _edge_accumulate(64, 2)(g2, row_p, col_p, ew_p)
  h3, asrc, adst = _row_call(
      _gat_mm_kernel,
      [_rb(64), _rb(64), _rb(64), _rb(1), _full(1, 64), _full(64, 64),
       _full(64, 1), _full(64, 1)],
      (jax.ShapeDtypeStruct((n, 64), jnp.float32),
       jax.ShapeDtypeStruct((n, 1), jnp.float32),
       jax.ShapeDtypeStruct((n, 1), jnp.float32)),
      (_rb(64), _rb(1), _rb(1)))(
          acc2[0, :n], acc2[1, :n], h2p, dinv, b2.reshape(1, 64), W3,
          att_src.reshape(64, 1), att_dst.reshape(64, 1))

  pq, exs = pl.pallas_call(
      _att_prep_kernel,
      out_shape=(jax.ShapeDtypeStruct((n, 2), jnp.float32),
                 jax.ShapeDtypeStruct((n, 1), jnp.float32)))(asrc, adst)

  asx = jnp.concatenate([asrc.reshape(n), jnp.full((16,), -1e30, jnp.float32)])
  h3x = jnp.concatenate([h3, jnp.zeros((16, 64), jnp.float32)])
  s_parts, acc3 = _gat_edge_kernel(h3x, asx, pq, row_q, col_p)

  out = _row_call(
      _final_kernel,
      [_rb(64), _rb(64), _rb(1), _rb(1), _rb(1), _rb(64), _full(1, 64)],
      jax.ShapeDtypeStruct((n, 64), jnp.float32),
      _rb(64))(
          acc3[0, :n], acc3[1, :n], s_parts[0, :n].reshape(n, 1),
          s_parts[1, :n].reshape(n, 1), exs, h3, b3.reshape(1, 64))
  return out


# speedup vs baseline: 24.9030x; 1.1912x over previous
import dataclasses
import functools

import jax
import jax.numpy as jnp
from jax import lax
from jax.experimental import pallas as pl
from jax.experimental.pallas import tpu as pltpu
from jax.experimental.pallas import tpu_sc as plsc

N_NODES = 10000
N_EDGES = 320000
NP = 10240
N_WORKERS = 32
CH = 128
EPW = 10240
EPAD = EPW * N_WORKERS
ROWS_PER_TILE = NP // 16
CHUNKS_PER_TILE = ROWS_PER_TILE // CH

_MESH = plsc.VectorSubcoreMesh(core_axis_name="c", subcore_axis_name="s")

_SC_PARAMS = pltpu.CompilerParams()
if "needs_layout_passes" in pltpu.CompilerParams.__dataclass_fields__:
  _SC_PARAMS = dataclasses.replace(_SC_PARAMS, needs_layout_passes=False)
_SC_PARAMS_LINEAR = dataclasses.replace(_SC_PARAMS, use_tc_tiling_on_sc=False)


def _edge_accumulate(d_feat, sub, staged):
  CHUNK = CH * sub
  NCH = EPW // CHUNK

  scratch = [
      pltpu.VMEM((2, sub, CH), jnp.int32),
      pltpu.VMEM((2, sub, CH), jnp.int32),
      pltpu.VMEM((2, sub, CH), jnp.float32),
      pltpu.VMEM((CHUNK, d_feat), jnp.float32),
      pltpu.VMEM_SHARED((NP, d_feat), jnp.float32),
      pltpu.SemaphoreType.DMA,
      pltpu.SemaphoreType.DMA,
  ]
  if staged:
    scratch.append(pltpu.VMEM_SHARED((NP, d_feat), jnp.float32))

  @functools.partial(
      pl.kernel,
      out_type=jax.ShapeDtypeStruct((2, NP, d_feat), jnp.float32),
      mesh=_MESH,
      compiler_params=_SC_PARAMS if d_feat == 128 else _SC_PARAMS_LINEAR,
      scratch_types=scratch,
  )
  def k(src_hbm, row_hbm, col_hbm, w_hbm, out_hbm, row_v, col_v, w_v,
        rows_v, acc_sh, si0, si1, *maybe_src_sh):
    cid = lax.axis_index("c")
    sid = lax.axis_index("s")
    wid = cid * 16 + sid
    si = (si0, si1)
    src = maybe_src_sh[0] if staged else src_hbm
    if staged:
      pltpu.sync_copy(src_hbm.at[pl.ds(sid * ROWS_PER_TILE, ROWS_PER_TILE)],
                      maybe_src_sh[0].at[pl.ds(sid * ROWS_PER_TILE,
                                               ROWS_PER_TILE)])

    def start_idx(chunk, b):
      base = wid * EPW + chunk * CHUNK
      for s in range(sub):
        pltpu.async_copy(row_hbm.at[pl.ds(base + s * CH, CH)],
                         row_v.at[b, s], si[b])
        pltpu.async_copy(col_hbm.at[pl.ds(base + s * CH, CH)],
                         col_v.at[b, s], si[b])
        pltpu.async_copy(w_hbm.at[pl.ds(base + s * CH, CH)],
                         w_v.at[b, s], si[b])

    def wait_idx(b):
      for s in range(sub):
        pltpu.make_async_copy(row_hbm.at[pl.ds(0, CH)], row_v.at[b, s],
                              si[b]).wait()
        pltpu.make_async_copy(col_hbm.at[pl.ds(0, CH)], col_v.at[b, s],
                              si[b]).wait()
        pltpu.make_async_copy(w_hbm.at[pl.ds(0, CH)], w_v.at[b, s],
                              si[b]).wait()

    @pl.loop(0, CH)
    def _(i):
      for d in range(d_feat // 16):
        rows_v[i, pl.ds(d * 16, 16)] = jnp.zeros((16,), jnp.float32)

    @pl.loop(0, CHUNKS_PER_TILE)
    def _(j):
      pltpu.sync_copy(rows_v.at[pl.ds(0, CH)],
                      acc_sh.at[pl.ds(sid * ROWS_PER_TILE + j * CH, CH)])

    plsc.subcore_barrier()

    start_idx(0, 0)
    start_idx(1, 1)

    @pl.loop(0, NCH // 2)
    def _(j):
      for b in (0, 1):
        k = 2 * j + b
        wait_idx(b)

        for s in range(sub):
          pltpu.sync_copy(src.at[row_v.at[b, s]],
                          rows_v.at[pl.ds(s * CH, CH)])

        for s in range(sub):
          @pl.loop(0, CH)
          def _(i):
            w = plsc.load_gather(w_v.at[b, s], [jnp.full((16,), i, jnp.int32)])
            for d in range(d_feat // 16):
              sl = (s * CH + i, pl.ds(d * 16, 16))
              rows_v[sl] = rows_v[sl] * w

        for s in range(sub):
          pltpu.sync_copy(rows_v.at[pl.ds(s * CH, CH)],
                          acc_sh.at[col_v.at[b, s]], add=True)

        nk = jnp.where(k + 2 >= NCH, k + 2 - NCH, k + 2)
        start_idx(nk, b)

    wait_idx(0)
    wait_idx(1)

    plsc.subcore_barrier()

    @pl.loop(0, CHUNKS_PER_TILE)
    def _(j):
      start = sid * ROWS_PER_TILE + j * CH
      pltpu.sync_copy(acc_sh.at[pl.ds(start, CH)],
                      out_hbm.at[cid, pl.ds(start, CH)])

  return k


_DEG_SUB = 4


@functools.partial(
    pl.kernel,
    out_type=jax.ShapeDtypeStruct((2, NP), jnp.float32),
    mesh=_MESH,
    compiler_params=_SC_PARAMS,
    scratch_types=[
        pltpu.VMEM((2, _DEG_SUB, CH), jnp.int32),
        pltpu.VMEM((2, _DEG_SUB, CH), jnp.float32),
        pltpu.VMEM_SHARED((NP,), jnp.float32),
        pltpu.SemaphoreType.DMA,
        pltpu.SemaphoreType.DMA,
    ],
)
def _degree_kernel(col_hbm, w_hbm, out_hbm, col_v, w_v, deg_sh, si0, si1):
  cid = lax.axis_index("c")
  sid = lax.axis_index("s")
  wid = cid * 16 + sid
  si = (si0, si1)
  chunk = _DEG_SUB * CH
  nch = EPW // chunk

  def start_idx(k, b):
    base = wid * EPW + k * chunk
    for s in range(_DEG_SUB):
      pltpu.async_copy(col_hbm.at[pl.ds(base + s * CH, CH)], col_v.at[b, s],
                       si[b])
      pltpu.async_copy(w_hbm.at[pl.ds(base + s * CH, CH)], w_v.at[b, s],
                       si[b])

  def wait_idx(b):
    for s in range(_DEG_SUB):
      pltpu.make_async_copy(col_hbm.at[pl.ds(0, CH)], col_v.at[b, s],
                            si[b]).wait()
      pltpu.make_async_copy(w_hbm.at[pl.ds(0, CH)], w_v.at[b, s],
                            si[b]).wait()

  @pl.loop(0, CH // 16)
  def _(g):
    w_v[0, 0, pl.ds(g * 16, 16)] = jnp.zeros((16,), jnp.float32)

  @pl.loop(0, CHUNKS_PER_TILE)
  def _(j):
    pltpu.sync_copy(w_v.at[0, 0],
                    deg_sh.at[pl.ds(sid * ROWS_PER_TILE + j * CH, CH)])

  plsc.subcore_barrier()
  start_idx(0, 0)
  start_idx(1, 1)

  @pl.loop(0, nch // 2)
  def _(j):
    for b in (0, 1):
      k = 2 * j + b
      wait_idx(b)
      for s in range(_DEG_SUB):
        pltpu.sync_copy(w_v.at[b, s], deg_sh.at[col_v.at[b, s]], add=True)
      nk = jnp.where(k + 2 >= nch, k + 2 - nch, k + 2)
      start_idx(nk, b)

  wait_idx(0)
  wait_idx(1)
  plsc.subcore_barrier()

  @pl.loop(0, CHUNKS_PER_TILE)
  def _(j):
    start = sid * ROWS_PER_TILE + j * CH
    pltpu.sync_copy(deg_sh.at[pl.ds(start, CH)], out_hbm.at[cid, pl.ds(start, CH)])


_GAT_SUB = 2
_GAT_CHUNK = _GAT_SUB * CH


@functools.partial(
    pl.kernel,
    out_type=[
        jax.ShapeDtypeStruct((2, NP), jnp.float32),
        jax.ShapeDtypeStruct((2, NP, 64), jnp.float32),
    ],
    mesh=_MESH,
    compiler_params=_SC_PARAMS_LINEAR,
    scratch_types=[
        pltpu.VMEM((2, _GAT_SUB, CH), jnp.int32),
        pltpu.VMEM((2, _GAT_SUB, CH), jnp.int32),
        pltpu.VMEM((_GAT_CHUNK,), jnp.float32),
        pltpu.VMEM((_GAT_CHUNK, 2), jnp.float32),
        pltpu.VMEM((_GAT_CHUNK,), jnp.float32),
        pltpu.VMEM((_GAT_CHUNK, 64), jnp.float32),
        pltpu.VMEM_SHARED((NP,), jnp.float32),
        pltpu.VMEM_SHARED((NP, 64), jnp.float32),
        pltpu.VMEM_SHARED((NP, 64), jnp.float32),
        pltpu.VMEM_SHARED((NP,), jnp.float32),
        pltpu.VMEM_SHARED((NP, 2), jnp.float32),
        pltpu.SemaphoreType.DMA,
        pltpu.SemaphoreType.DMA,
    ],
)
def _gat_edge_kernel(h3_hbm, asrc_hbm, pq_hbm, row_hbm, col_hbm,
                     s_out, acc_out, row_v, col_v, as_v, pq_v, ex_v, rows_v,
                     s_sh, acc_sh, h3_sh, as_sh, pq_sh, si0, si1):
  cid = lax.axis_index("c")
  sid = lax.axis_index("s")
  wid = cid * 16 + sid
  si = (si0, si1)
  nch = EPW // _GAT_CHUNK

  def start_idx(k, b):
    base = wid * EPW + k * _GAT_CHUNK
    for s in range(_GAT_SUB):
      pltpu.async_copy(row_hbm.at[pl.ds(base + s * CH, CH)], row_v.at[b, s],
                       si[b])
      pltpu.async_copy(col_hbm.at[pl.ds(base + s * CH, CH)], col_v.at[b, s],
                       si[b])

  def wait_idx(b):
    for s in range(_GAT_SUB):
      pltpu.make_async_copy(row_hbm.at[pl.ds(0, CH)], row_v.at[b, s],
                            si[b]).wait()
      pltpu.make_async_copy(col_hbm.at[pl.ds(0, CH)], col_v.at[b, s],
                            si[b]).wait()

  @pl.loop(0, CH)
  def _(i):
    for d in range(4):
      rows_v[i, pl.ds(d * 16, 16)] = jnp.zeros((16,), jnp.float32)

  @pl.loop(0, CH // 16)
  def _(g):
    ex_v[pl.ds(g * 16, 16)] = jnp.zeros((16,), jnp.float32)

  @pl.loop(0, CHUNKS_PER_TILE)
  def _(j):
    start = sid * ROWS_PER_TILE + j * CH
    pltpu.sync_copy(rows_v.at[pl.ds(0, CH)], acc_sh.at[pl.ds(start, CH)])
    pltpu.sync_copy(ex_v.at[pl.ds(0, CH)], s_sh.at[pl.ds(start, CH)])

  tile = pl.ds(sid * ROWS_PER_TILE, ROWS_PER_TILE)
  pltpu.sync_copy(h3_hbm.at[tile], h3_sh.at[tile])
  pltpu.sync_copy(asrc_hbm.at[tile], as_sh.at[tile])
  pltpu.sync_copy(pq_hbm.at[tile], pq_sh.at[tile])

  plsc.subcore_barrier()
  start_idx(0, 0)
  start_idx(1, 1)

  @pl.loop(0, nch // 2)
  def _(j):
    for b in (0, 1):
      k = 2 * j + b
      wait_idx(b)

      for s in range(_GAT_SUB):
        pltpu.sync_copy(as_sh.at[row_v.at[b, s]],
                        as_v.at[pl.ds(s * CH, CH)])
        pltpu.sync_copy(pq_sh.at[col_v.at[b, s]],
                        pq_v.at[pl.ds(s * CH, CH)])
        pltpu.sync_copy(h3_sh.at[row_v.at[b, s]],
                        rows_v.at[pl.ds(s * CH, CH)])

      @pl.loop(0, _GAT_CHUNK // 16)
      def _(g):
        lane = lax.iota(jnp.int32, 16) + g * 16
        ad = plsc.load_gather(pq_v, [lane, jnp.zeros((16,), jnp.int32)])
        off = plsc.load_gather(pq_v, [lane, jnp.ones((16,), jnp.int32)])
        sl = pl.ds(g * 16, 16)
        z = as_v[sl] + ad
        e = jnp.where(z > 0.0, z, 0.2 * z)
        ex_v[sl] = jnp.exp(e - off)

      @pl.loop(0, _GAT_CHUNK)
      def _(i):
        w = plsc.load_gather(ex_v, [jnp.full((16,), i, jnp.int32)])
        for d in range(4):
          sl = (i, pl.ds(d * 16, 16))
          rows_v[sl] = rows_v[sl] * w

      for s in range(_GAT_SUB):
        pltpu.sync_copy(ex_v.at[pl.ds(s * CH, CH)],
                        s_sh.at[col_v.at[b, s]], add=True)
        pltpu.sync_copy(rows_v.at[pl.ds(s * CH, CH)],
                        acc_sh.at[col_v.at[b, s]], add=True)

      nk = jnp.where(k + 2 >= nch, k + 2 - nch, k + 2)
      start_idx(nk, b)

  wait_idx(0)
  wait_idx(1)
  plsc.subcore_barrier()

  @pl.loop(0, CHUNKS_PER_TILE)
  def _(j):
    start = sid * ROWS_PER_TILE + j * CH
    pltpu.sync_copy(s_sh.at[pl.ds(start, CH)], s_out.at[cid, pl.ds(start, CH)])
    pltpu.sync_copy(acc_sh.at[pl.ds(start, CH)],
                    acc_out.at[cid, pl.ds(start, CH)])


BR = 2000
GRID = N_NODES // BR


def _rb(d):
  return pl.BlockSpec((BR, d), lambda i: (i, 0))


def _full(s0, s1):
  return pl.BlockSpec((s0, s1), lambda i: (0, 0))


def _row_call(body, in_specs, out_shape, out_specs):
  return pl.pallas_call(body, grid=(GRID,), in_specs=in_specs,
                        out_shape=out_shape, out_specs=out_specs)


_DOT = functools.partial(jnp.dot, preferred_element_type=jnp.float32,
                         precision=lax.Precision.HIGHEST)


def _mm_kernel(x_ref, w_ref, o_ref):
  o_ref[...] = _DOT(x_ref[...], w_ref[...])


def _scale_kernel(hp_ref, d0_ref, d1_ref, g_ref, dinv_ref):
  deg = d0_ref[...] + d1_ref[...] + 1.0
  dinv = lax.rsqrt(deg)
  dinv_ref[...] = dinv
  g_ref[...] = hp_ref[...] * dinv


def _combine_kernel(a0_ref, a1_ref, hp_ref, dinv_ref, b_ref, w_ref,
                    hnext_ref, gnext_ref):
  dinv = dinv_ref[...]
  out = dinv * (a0_ref[...] + a1_ref[...]) + dinv * dinv * hp_ref[...] \
      + b_ref[...]
  h = out * (1.0 / (1.0 + jnp.exp(-out)))
  hp = _DOT(h, w_ref[...])
  hnext_ref[...] = hp
  gnext_ref[...] = hp * dinv


def _gat_mm_kernel(a0_ref, a1_ref, hp_ref, dinv_ref, b_ref, w_ref,
                   atts_ref, attd_ref, h3_ref, asrc_ref, adst_ref):
  dinv = dinv_ref[...]
  out = dinv * (a0_ref[...] + a1_ref[...]) + dinv * dinv * hp_ref[...] \
      + b_ref[...]
  h2 = out * (1.0 / (1.0 + jnp.exp(-out)))
  h3 = _DOT(h2, w_ref[...])
  h3_ref[...] = h3
  asrc_ref[...] = _DOT(h3, atts_ref[...])
  adst_ref[...] = _DOT(h3, attd_ref[...])


def _att_prep_kernel(asrc_ref, adst_ref, pq_ref, exs_ref):
  asrc = asrc_ref[...]
  adst = adst_ref[...]
  amax = jnp.max(asrc)
  es = asrc + adst
  e_self = jnp.where(es > 0.0, es, 0.2 * es)
  off = jnp.maximum(e_self, adst + amax)
  pq_ref[...] = jnp.concatenate([adst, off], axis=1)
  exs_ref[...] = jnp.exp(e_self - off)


def _final_kernel(a0_ref, a1_ref, s0_ref, s1_ref, exs_ref, h3_ref, b_ref,
                  o_ref):
  s = s0_ref[...] + s1_ref[...] + exs_ref[...]
  num = a0_ref[...] + a1_ref[...] + exs_ref[...] * h3_ref[...]
  o3 = num / s + b_ref[...]
  m = jnp.max(o3, axis=1, keepdims=True)
  e = jnp.exp(o3 - m)
  o_ref[...] = e / jnp.sum(e, axis=1, keepdims=True)


def kernel(x, edge_index, edge_weight, W1, b1, W2, b2, W3, att_src, att_dst,
           b3):
  n = N_NODES
  row, col = edge_index[0], edge_index[1]

  pad = EPAD - N_EDGES
  pad_idx = (jnp.arange(pad, dtype=jnp.int32) * 8) % n
  row_p = jnp.concatenate([row, pad_idx])
  col_p = jnp.concatenate([col, pad_idx])
  ew_p = jnp.concatenate([edge_weight, jnp.zeros((pad,), jnp.float32)])
  row_q = jnp.concatenate(
      [row, n + (jnp.arange(pad, dtype=jnp.int32) % 16)])

  deg_parts = _degree_kernel(col_p, ew_p)
  h1p = _row_call(_mm_kernel, [_rb(128), _full(128, 128)],
                  jax.ShapeDtypeStruct((n, 128), jnp.float32),
                  _rb(128))(x, W1)

  d0 = deg_parts[0, :n].reshape(n, 1)
  d1 = deg_parts[1, :n].reshape(n, 1)
  g1, dinv = _row_call(
      _scale_kernel, [_rb(128), _rb(1), _rb(1)],
      (jax.ShapeDtypeStruct((n, 128), jnp.float32),
       jax.ShapeDtypeStruct((n, 1), jnp.float32)),
      (_rb(128), _rb(1)))(h1p, d0, d1)

  acc1 = _edge_accumulate(128, 2, False)(g1, row_p, col_p, ew_p)
  h2p, g2 = _row_call(
      _combine_kernel,
      [_rb(128), _rb(128), _rb(128), _rb(1), _full(1, 128), _full(128, 64)],
      (jax.ShapeDtypeStruct((n, 64), jnp.float32),
       jax.ShapeDtypeStruct((n, 64), jnp.float32)),
      (_rb(64), _rb(64)))(
          acc1[0, :n], acc1[1, :n], h1p, dinv, b1.reshape(1, 128), W2)

  g2x = jnp.concatenate([g2, jnp.zeros((NP - n, 64), jnp.float32)])
  acc2 = _edge_accumulate(64, 2, True)(g2x, row_p, col_p, ew_p)
  h3, asrc, adst = _row_call(
      _gat_mm_kernel,
      [_rb(64), _rb(64), _rb(64), _rb(1), _full(1, 64), _full(64, 64),
       _full(64, 1), _full(64, 1)],
      (jax.ShapeDtypeStruct((n, 64), jnp.float32),
       jax.ShapeDtypeStruct((n, 1), jnp.float32),
       jax.ShapeDtypeStruct((n, 1), jnp.float32)),
      (_rb(64), _rb(1), _rb(1)))(
          acc2[0, :n], acc2[1, :n], h2p, dinv, b2.reshape(1, 64), W3,
          att_src.reshape(64, 1), att_dst.reshape(64, 1))

  pq, exs = pl.pallas_call(
      _att_prep_kernel,
      out_shape=(jax.ShapeDtypeStruct((n, 2), jnp.float32),
                 jax.ShapeDtypeStruct((n, 1), jnp.float32)))(asrc, adst)

  asx = jnp.concatenate(
      [asrc.reshape(n), jnp.full((NP - n,), -1e30, jnp.float32)])
  h3x = jnp.concatenate([h3, jnp.zeros((NP - n, 64), jnp.float32)])
  pqx = jnp.concatenate([pq, jnp.zeros((NP - n, 2), jnp.float32)])
  s_parts, acc3 = _gat_edge_kernel(h3x, asx, pqx, row_q, col_p)

  out = _row_call(
      _final_kernel,
      [_rb(64), _rb(64), _rb(1), _rb(1), _rb(1), _rb(64), _full(1, 64)],
      jax.ShapeDtypeStruct((n, 64), jnp.float32),
      _rb(64))(
          acc3[0, :n], acc3[1, :n], s_parts[0, :n].reshape(n, 1),
          s_parts[1, :n].reshape(n, 1), exs, h3, b3.reshape(1, 64))
  return out


# speedup vs baseline: 25.4428x; 1.0217x over previous
import dataclasses
import functools

import jax
import jax.numpy as jnp
from jax import lax
from jax.experimental import pallas as pl
from jax.experimental.pallas import tpu as pltpu
from jax.experimental.pallas import tpu_sc as plsc

N_NODES = 10000
N_EDGES = 320000
NP = 10240
N_WORKERS = 32
CH = 128
EPW = 10240
EPAD = EPW * N_WORKERS
ROWS_PER_TILE = NP // 16
CHUNKS_PER_TILE = ROWS_PER_TILE // CH

_MESH = plsc.VectorSubcoreMesh(core_axis_name="c", subcore_axis_name="s")

_SC_PARAMS = pltpu.CompilerParams()
if "needs_layout_passes" in pltpu.CompilerParams.__dataclass_fields__:
  _SC_PARAMS = dataclasses.replace(_SC_PARAMS, needs_layout_passes=False)
_SC_PARAMS_LINEAR = dataclasses.replace(_SC_PARAMS, use_tc_tiling_on_sc=False)


def _edge_accumulate(d_feat, sub, staged):
  CHUNK = CH * sub
  NCH = EPW // CHUNK

  scratch = [
      pltpu.VMEM((2, sub, CH), jnp.int32),
      pltpu.VMEM((2, sub, CH), jnp.int32),
      pltpu.VMEM((2, sub, CH), jnp.float32),
      pltpu.VMEM((CHUNK, d_feat), jnp.float32),
      pltpu.VMEM_SHARED((NP, d_feat), jnp.float32),
      pltpu.SemaphoreType.DMA,
      pltpu.SemaphoreType.DMA,
  ]
  if staged:
    scratch.append(pltpu.VMEM_SHARED((NP, d_feat), jnp.float32))

  @functools.partial(
      pl.kernel,
      out_type=jax.ShapeDtypeStruct((2, NP, d_feat), jnp.float32),
      mesh=_MESH,
      compiler_params=_SC_PARAMS if d_feat == 128 else _SC_PARAMS_LINEAR,
      scratch_types=scratch,
  )
  def k(src_hbm, row_hbm, col_hbm, w_hbm, out_hbm, row_v, col_v, w_v,
        rows_v, acc_sh, si0, si1, *maybe_src_sh):
    cid = lax.axis_index("c")
    sid = lax.axis_index("s")
    wid = cid * 16 + sid
    si = (si0, si1)
    src = maybe_src_sh[0] if staged else src_hbm
    if staged:
      pltpu.sync_copy(src_hbm.at[pl.ds(sid * ROWS_PER_TILE, ROWS_PER_TILE)],
                      maybe_src_sh[0].at[pl.ds(sid * ROWS_PER_TILE,
                                               ROWS_PER_TILE)])

    def start_idx(chunk, b):
      base = wid * EPW + chunk * CHUNK
      for s in range(sub):
        pltpu.async_copy(row_hbm.at[pl.ds(base + s * CH, CH)],
                         row_v.at[b, s], si[b])
        pltpu.async_copy(col_hbm.at[pl.ds(base + s * CH, CH)],
                         col_v.at[b, s], si[b])
        pltpu.async_copy(w_hbm.at[pl.ds(base + s * CH, CH)],
                         w_v.at[b, s], si[b])

    def wait_idx(b):
      for s in range(sub):
        pltpu.make_async_copy(row_hbm.at[pl.ds(0, CH)], row_v.at[b, s],
                              si[b]).wait()
        pltpu.make_async_copy(col_hbm.at[pl.ds(0, CH)], col_v.at[b, s],
                              si[b]).wait()
        pltpu.make_async_copy(w_hbm.at[pl.ds(0, CH)], w_v.at[b, s],
                              si[b]).wait()

    @pl.loop(0, CH)
    def _(i):
      for d in range(d_feat // 16):
        rows_v[i, pl.ds(d * 16, 16)] = jnp.zeros((16,), jnp.float32)

    @pl.loop(0, CHUNKS_PER_TILE)
    def _(j):
      pltpu.sync_copy(rows_v.at[pl.ds(0, CH)],
                      acc_sh.at[pl.ds(sid * ROWS_PER_TILE + j * CH, CH)])

    plsc.subcore_barrier()

    start_idx(0, 0)
    start_idx(1, 1)

    @pl.loop(0, NCH // 2)
    def _(j):
      for b in (0, 1):
        k = 2 * j + b
        wait_idx(b)

        for s in range(sub):
          pltpu.sync_copy(src.at[row_v.at[b, s]],
                          rows_v.at[pl.ds(s * CH, CH)])

        for s in range(sub):
          @pl.loop(0, CH)
          def _(i):
            w = plsc.load_gather(w_v.at[b, s], [jnp.full((16,), i, jnp.int32)])
            for d in range(d_feat // 16):
              sl = (s * CH + i, pl.ds(d * 16, 16))
              rows_v[sl] = rows_v[sl] * w

        for s in range(sub):
          pltpu.sync_copy(rows_v.at[pl.ds(s * CH, CH)],
                          acc_sh.at[col_v.at[b, s]], add=True)

        nk = jnp.where(k + 2 >= NCH, k + 2 - NCH, k + 2)
        start_idx(nk, b)

    wait_idx(0)
    wait_idx(1)

    plsc.subcore_barrier()

    @pl.loop(0, CHUNKS_PER_TILE)
    def _(j):
      start = sid * ROWS_PER_TILE + j * CH
      pltpu.sync_copy(acc_sh.at[pl.ds(start, CH)],
                      out_hbm.at[cid, pl.ds(start, CH)])

  return k


_DEG_SUB = 4


@functools.partial(
    pl.kernel,
    out_type=jax.ShapeDtypeStruct((2, NP), jnp.float32),
    mesh=_MESH,
    compiler_params=_SC_PARAMS,
    scratch_types=[
        pltpu.VMEM((2, _DEG_SUB, CH), jnp.int32),
        pltpu.VMEM((2, _DEG_SUB, CH), jnp.float32),
        pltpu.VMEM_SHARED((NP,), jnp.float32),
        pltpu.SemaphoreType.DMA,
        pltpu.SemaphoreType.DMA,
    ],
)
def _degree_kernel(col_hbm, w_hbm, out_hbm, col_v, w_v, deg_sh, si0, si1):
  cid = lax.axis_index("c")
  sid = lax.axis_index("s")
  wid = cid * 16 + sid
  si = (si0, si1)
  chunk = _DEG_SUB * CH
  nch = EPW // chunk

  def start_idx(k, b):
    base = wid * EPW + k * chunk
    for s in range(_DEG_SUB):
      pltpu.async_copy(col_hbm.at[pl.ds(base + s * CH, CH)], col_v.at[b, s],
                       si[b])
      pltpu.async_copy(w_hbm.at[pl.ds(base + s * CH, CH)], w_v.at[b, s],
                       si[b])

  def wait_idx(b):
    for s in range(_DEG_SUB):
      pltpu.make_async_copy(col_hbm.at[pl.ds(0, CH)], col_v.at[b, s],
                            si[b]).wait()
      pltpu.make_async_copy(w_hbm.at[pl.ds(0, CH)], w_v.at[b, s],
                            si[b]).wait()

  @pl.loop(0, CH // 16)
  def _(g):
    w_v[0, 0, pl.ds(g * 16, 16)] = jnp.zeros((16,), jnp.float32)

  @pl.loop(0, CHUNKS_PER_TILE)
  def _(j):
    pltpu.sync_copy(w_v.at[0, 0],
                    deg_sh.at[pl.ds(sid * ROWS_PER_TILE + j * CH, CH)])

  plsc.subcore_barrier()
  start_idx(0, 0)
  start_idx(1, 1)

  @pl.loop(0, nch // 2)
  def _(j):
    for b in (0, 1):
      k = 2 * j + b
      wait_idx(b)
      for s in range(_DEG_SUB):
        pltpu.sync_copy(w_v.at[b, s], deg_sh.at[col_v.at[b, s]], add=True)
      nk = jnp.where(k + 2 >= nch, k + 2 - nch, k + 2)
      start_idx(nk, b)

  wait_idx(0)
  wait_idx(1)
  plsc.subcore_barrier()

  @pl.loop(0, CHUNKS_PER_TILE)
  def _(j):
    start = sid * ROWS_PER_TILE + j * CH
    pltpu.sync_copy(deg_sh.at[pl.ds(start, CH)], out_hbm.at[cid, pl.ds(start, CH)])


_GAT_SUB = 2
_GAT_CHUNK = _GAT_SUB * CH


@functools.partial(
    pl.kernel,
    out_type=[
        jax.ShapeDtypeStruct((2, NP), jnp.float32),
        jax.ShapeDtypeStruct((2, NP, 64), jnp.float32),
    ],
    mesh=_MESH,
    compiler_params=_SC_PARAMS_LINEAR,
    scratch_types=[
        pltpu.VMEM((2, _GAT_SUB, CH), jnp.int32),
        pltpu.VMEM((2, _GAT_SUB, CH), jnp.int32),
        pltpu.VMEM((_GAT_CHUNK,), jnp.float32),
        pltpu.VMEM((_GAT_CHUNK, 2), jnp.float32),
        pltpu.VMEM((_GAT_CHUNK,), jnp.float32),
        pltpu.VMEM((_GAT_CHUNK, 64), jnp.float32),
        pltpu.VMEM_SHARED((NP,), jnp.float32),
        pltpu.VMEM_SHARED((NP, 64), jnp.float32),
        pltpu.VMEM_SHARED((NP, 64), jnp.float32),
        pltpu.VMEM_SHARED((NP,), jnp.float32),
        pltpu.VMEM_SHARED((NP, 2), jnp.float32),
        pltpu.SemaphoreType.DMA,
        pltpu.SemaphoreType.DMA,
    ],
)
def _gat_edge_kernel(h3_hbm, asrc_hbm, pq_hbm, row_hbm, col_hbm,
                     s_out, acc_out, row_v, col_v, as_v, pq_v, ex_v, rows_v,
                     s_sh, acc_sh, h3_sh, as_sh, pq_sh, si0, si1):
  cid = lax.axis_index("c")
  sid = lax.axis_index("s")
  wid = cid * 16 + sid
  si = (si0, si1)
  nch = EPW // _GAT_CHUNK

  def start_idx(k, b):
    base = wid * EPW + k * _GAT_CHUNK
    for s in range(_GAT_SUB):
      pltpu.async_copy(row_hbm.at[pl.ds(base + s * CH, CH)], row_v.at[b, s],
                       si[b])
      pltpu.async_copy(col_hbm.at[pl.ds(base + s * CH, CH)], col_v.at[b, s],
                       si[b])

  def wait_idx(b):
    for s in range(_GAT_SUB):
      pltpu.make_async_copy(row_hbm.at[pl.ds(0, CH)], row_v.at[b, s],
                            si[b]).wait()
      pltpu.make_async_copy(col_hbm.at[pl.ds(0, CH)], col_v.at[b, s],
                            si[b]).wait()

  @pl.loop(0, CH)
  def _(i):
    for d in range(4):
      rows_v[i, pl.ds(d * 16, 16)] = jnp.zeros((16,), jnp.float32)

  @pl.loop(0, CH // 16)
  def _(g):
    ex_v[pl.ds(g * 16, 16)] = jnp.zeros((16,), jnp.float32)

  @pl.loop(0, CHUNKS_PER_TILE)
  def _(j):
    start = sid * ROWS_PER_TILE + j * CH
    pltpu.sync_copy(rows_v.at[pl.ds(0, CH)], acc_sh.at[pl.ds(start, CH)])
    pltpu.sync_copy(ex_v.at[pl.ds(0, CH)], s_sh.at[pl.ds(start, CH)])

  tile = pl.ds(sid * ROWS_PER_TILE, ROWS_PER_TILE)
  pltpu.sync_copy(h3_hbm.at[tile], h3_sh.at[tile])
  pltpu.sync_copy(asrc_hbm.at[tile], as_sh.at[tile])
  pltpu.sync_copy(pq_hbm.at[tile], pq_sh.at[tile])

  plsc.subcore_barrier()
  start_idx(0, 0)
  start_idx(1, 1)

  @pl.loop(0, nch // 2)
  def _(j):
    for b in (0, 1):
      k = 2 * j + b
      wait_idx(b)

      for s in range(_GAT_SUB):
        pltpu.sync_copy(as_sh.at[row_v.at[b, s]],
                        as_v.at[pl.ds(s * CH, CH)])
        pltpu.sync_copy(pq_sh.at[col_v.at[b, s]],
                        pq_v.at[pl.ds(s * CH, CH)])
        pltpu.sync_copy(h3_sh.at[row_v.at[b, s]],
                        rows_v.at[pl.ds(s * CH, CH)])

      @pl.loop(0, _GAT_CHUNK // 16)
      def _(g):
        lane = lax.iota(jnp.int32, 16) + g * 16
        ad = plsc.load_gather(pq_v, [lane, jnp.zeros((16,), jnp.int32)])
        off = plsc.load_gather(pq_v, [lane, jnp.ones((16,), jnp.int32)])
        sl = pl.ds(g * 16, 16)
        z = as_v[sl] + ad
        e = jnp.where(z > 0.0, z, 0.2 * z)
        ex_v[sl] = jnp.exp(e - off)

      @pl.loop(0, _GAT_CHUNK)
      def _(i):
        w = plsc.load_gather(ex_v, [jnp.full((16,), i, jnp.int32)])
        for d in range(4):
          sl = (i, pl.ds(d * 16, 16))
          rows_v[sl] = rows_v[sl] * w

      for s in range(_GAT_SUB):
        pltpu.sync_copy(ex_v.at[pl.ds(s * CH, CH)],
                        s_sh.at[col_v.at[b, s]], add=True)
        pltpu.sync_copy(rows_v.at[pl.ds(s * CH, CH)],
                        acc_sh.at[col_v.at[b, s]], add=True)

      nk = jnp.where(k + 2 >= nch, k + 2 - nch, k + 2)
      start_idx(nk, b)

  wait_idx(0)
  wait_idx(1)
  plsc.subcore_barrier()

  @pl.loop(0, CHUNKS_PER_TILE)
  def _(j):
    start = sid * ROWS_PER_TILE + j * CH
    pltpu.sync_copy(s_sh.at[pl.ds(start, CH)], s_out.at[cid, pl.ds(start, CH)])
    pltpu.sync_copy(acc_sh.at[pl.ds(start, CH)],
                    acc_out.at[cid, pl.ds(start, CH)])


BR = 2000
GRID = N_NODES // BR


def _rb(d):
  return pl.BlockSpec((BR, d), lambda i: (i, 0))


def _full(s0, s1):
  return pl.BlockSpec((s0, s1), lambda i: (0, 0))


def _part(core, d):
  return pl.BlockSpec((1, BR, d), lambda i, c=core: (c, i, 0))


def _row_call(body, in_specs, out_shape, out_specs):
  return pl.pallas_call(body, grid=(GRID,), in_specs=in_specs,
                        out_shape=out_shape, out_specs=out_specs)


_DOT = functools.partial(jnp.dot, preferred_element_type=jnp.float32,
                         precision=lax.Precision.HIGHEST)


def _mm_kernel(x_ref, w_ref, o_ref):
  o_ref[...] = _DOT(x_ref[...], w_ref[...])


def _scale_kernel(hp_ref, d0_ref, d1_ref, g_ref, dinv_ref):
  deg = d0_ref[0] + d1_ref[0] + 1.0
  dinv = lax.rsqrt(deg)
  dinv_ref[...] = dinv
  g_ref[...] = hp_ref[...] * dinv


def _combine_kernel(a0_ref, a1_ref, hp_ref, dinv_ref, b_ref, w_ref,
                    hnext_ref, gnext_ref):
  dinv = dinv_ref[...]
  out = dinv * (a0_ref[0] + a1_ref[0]) + dinv * dinv * hp_ref[...] \
      + b_ref[...]
  h = out * (1.0 / (1.0 + jnp.exp(-out)))
  hp = _DOT(h, w_ref[...])
  hnext_ref[...] = hp
  gnext_ref[...] = hp * dinv


def _gat_mm_kernel(a0_ref, a1_ref, hp_ref, dinv_ref, b_ref, w_ref,
                   atts_ref, attd_ref, h3_ref, asrc_ref, adst_ref):
  dinv = dinv_ref[...]
  out = dinv * (a0_ref[0] + a1_ref[0]) + dinv * dinv * hp_ref[...] \
      + b_ref[...]
  h2 = out * (1.0 / (1.0 + jnp.exp(-out)))
  h3 = _DOT(h2, w_ref[...])
  h3_ref[...] = h3
  asrc_ref[...] = _DOT(h3, atts_ref[...])
  adst_ref[...] = _DOT(h3, attd_ref[...])


def _att_prep_kernel(asrc_ref, adst_ref, pq_ref, exs_ref):
  asrc = asrc_ref[...]
  adst = adst_ref[...]
  amax = jnp.max(asrc)
  es = asrc + adst
  e_self = jnp.where(es > 0.0, es, 0.2 * es)
  off = jnp.maximum(e_self, adst + amax)
  pq_ref[...] = jnp.concatenate([adst, off], axis=1)
  exs_ref[...] = jnp.exp(e_self - off)


def _final_kernel(a0_ref, a1_ref, s0_ref, s1_ref, exs_ref, h3_ref, b_ref,
                  o_ref):
  s = s0_ref[0] + s1_ref[0] + exs_ref[...]
  num = a0_ref[0] + a1_ref[0] + exs_ref[...] * h3_ref[...]
  o3 = num / s + b_ref[...]
  m = jnp.max(o3, axis=1, keepdims=True)
  e = jnp.exp(o3 - m)
  o_ref[...] = e / jnp.sum(e, axis=1, keepdims=True)


def kernel(x, edge_index, edge_weight, W1, b1, W2, b2, W3, att_src, att_dst,
           b3):
  n = N_NODES
  row, col = edge_index[0], edge_index[1]

  pad = EPAD - N_EDGES
  pad_idx = (jnp.arange(pad, dtype=jnp.int32) * 8) % n
  row_p = jnp.concatenate([row, pad_idx])
  col_p = jnp.concatenate([col, pad_idx])
  ew_p = jnp.concatenate([edge_weight, jnp.zeros((pad,), jnp.float32)])
  row_q = jnp.concatenate(
      [row, n + (jnp.arange(pad, dtype=jnp.int32) % 16)])

  deg_parts = _degree_kernel(col_p, ew_p)
  h1p = _row_call(_mm_kernel, [_rb(128), _full(128, 128)],
                  jax.ShapeDtypeStruct((n, 128), jnp.float32),
                  _rb(128))(x, W1)

  dp = deg_parts.reshape(2, NP, 1)
  g1, dinv = _row_call(
      _scale_kernel, [_rb(128), _part(0, 1), _part(1, 1)],
      (jax.ShapeDtypeStruct((n, 128), jnp.float32),
       jax.ShapeDtypeStruct((n, 1), jnp.float32)),
      (_rb(128), _rb(1)))(h1p, dp, dp)

  acc1 = _edge_accumulate(128, 2, False)(g1, row_p, col_p, ew_p)
  h2p, g2 = _row_call(
      _combine_kernel,
      [_part(0, 128), _part(1, 128), _rb(128), _rb(1), _full(1, 128),
       _full(128, 64)],
      (jax.ShapeDtypeStruct((n, 64), jnp.float32),
       jax.ShapeDtypeStruct((n, 64), jnp.float32)),
      (_rb(64), _rb(64)))(
          acc1, acc1, h1p, dinv, b1.reshape(1, 128), W2)

  g2x = jnp.concatenate([g2, jnp.zeros((NP - n, 64), jnp.float32)])
  acc2 = _edge_accumulate(64, 2, True)(g2x, row_p, col_p, ew_p)
  h3, asrc, adst = _row_call(
      _gat_mm_kernel,
      [_part(0, 64), _part(1, 64), _rb(64), _rb(1), _full(1, 64),
       _full(64, 64), _full(64, 1), _full(64, 1)],
      (jax.ShapeDtypeStruct((n, 64), jnp.float32),
       jax.ShapeDtypeStruct((n, 1), jnp.float32),
       jax.ShapeDtypeStruct((n, 1), jnp.float32)),
      (_rb(64), _rb(1), _rb(1)))(
          acc2, acc2, h2p, dinv, b2.reshape(1, 64), W3,
          att_src.reshape(64, 1), att_dst.reshape(64, 1))

  pq, exs = pl.pallas_call(
      _att_prep_kernel,
      out_shape=(jax.ShapeDtypeStruct((n, 2), jnp.float32),
                 jax.ShapeDtypeStruct((n, 1), jnp.float32)))(asrc, adst)

  asx = jnp.concatenate(
      [asrc.reshape(n), jnp.full((NP - n,), -1e30, jnp.float32)])
  h3x = jnp.concatenate([h3, jnp.zeros((NP - n, 64), jnp.float32)])
  pqx = jnp.concatenate([pq, jnp.zeros((NP - n, 2), jnp.float32)])
  s_parts, acc3 = _gat_edge_kernel(h3x, asx, pqx, row_q, col_p)

  sp = s_parts.reshape(2, NP, 1)
  out = _row_call(
      _final_kernel,
      [_part(0, 64), _part(1, 64), _part(0, 1), _part(1, 1), _rb(1), _rb(64),
       _full(1, 64)],
      jax.ShapeDtypeStruct((n, 64), jnp.float32),
      _rb(64))(
          acc3, acc3, sp, sp, exs, h3, b3.reshape(1, 64))
  return out


# speedup vs baseline: 29.6755x; 1.1664x over previous
import dataclasses
import functools

import jax
import jax.numpy as jnp
from jax import lax
from jax.experimental import pallas as pl
from jax.experimental.pallas import tpu as pltpu
from jax.experimental.pallas import tpu_sc as plsc

N_NODES = 10000
N_EDGES = 320000
NP = 10240
N_WORKERS = 32
CH = 128
EPW = 10240
EPAD = EPW * N_WORKERS
ROWS_PER_TILE = NP // 16
CHUNKS_PER_TILE = ROWS_PER_TILE // CH

_MESH = plsc.VectorSubcoreMesh(core_axis_name="c", subcore_axis_name="s")

_SC_PARAMS = pltpu.CompilerParams()
if "needs_layout_passes" in pltpu.CompilerParams.__dataclass_fields__:
  _SC_PARAMS = dataclasses.replace(_SC_PARAMS, needs_layout_passes=False)
_SC_PARAMS_LINEAR = dataclasses.replace(_SC_PARAMS, use_tc_tiling_on_sc=False)


def _edge_accumulate(d_feat, sub, staged):
  CHUNK = CH * sub
  NCH = EPW // CHUNK

  scratch = [
      pltpu.VMEM((2, sub, CH), jnp.int32),
      pltpu.VMEM((2, sub, CH), jnp.int32),
      pltpu.VMEM((2, sub, CH), jnp.float32),
      pltpu.VMEM((CHUNK, d_feat), jnp.float32),
      pltpu.VMEM_SHARED((NP, d_feat), jnp.float32),
      pltpu.SemaphoreType.DMA,
      pltpu.SemaphoreType.DMA,
  ]
  scratch += [pltpu.SemaphoreType.DMA] * (2 * sub)
  if staged:
    scratch.append(pltpu.VMEM_SHARED((NP, d_feat), jnp.float32))

  @functools.partial(
      pl.kernel,
      out_type=jax.ShapeDtypeStruct((2, NP, d_feat), jnp.float32),
      mesh=_MESH,
      compiler_params=_SC_PARAMS if d_feat == 128 else _SC_PARAMS_LINEAR,
      scratch_types=scratch,
  )
  def k(src_hbm, row_hbm, col_hbm, w_hbm, out_hbm, row_v, col_v, w_v,
        rows_v, acc_sh, si0, si1, *rest):
    cid = lax.axis_index("c")
    sid = lax.axis_index("s")
    wid = cid * 16 + sid
    si = (si0, si1)
    sg = rest[:sub]
    ss = rest[sub:2 * sub]
    src = rest[2 * sub] if staged else src_hbm
    if staged:
      pltpu.sync_copy(src_hbm.at[pl.ds(sid * ROWS_PER_TILE, ROWS_PER_TILE)],
                      src.at[pl.ds(sid * ROWS_PER_TILE, ROWS_PER_TILE)])

    def start_idx(chunk, b):
      base = wid * EPW + chunk * CHUNK
      for s in range(sub):
        pltpu.async_copy(row_hbm.at[pl.ds(base + s * CH, CH)],
                         row_v.at[b, s], si[b])
        pltpu.async_copy(col_hbm.at[pl.ds(base + s * CH, CH)],
                         col_v.at[b, s], si[b])
        pltpu.async_copy(w_hbm.at[pl.ds(base + s * CH, CH)],
                         w_v.at[b, s], si[b])

    def wait_idx(b):
      for s in range(sub):
        pltpu.make_async_copy(row_hbm.at[pl.ds(0, CH)], row_v.at[b, s],
                              si[b]).wait()
        pltpu.make_async_copy(col_hbm.at[pl.ds(0, CH)], col_v.at[b, s],
                              si[b]).wait()
        pltpu.make_async_copy(w_hbm.at[pl.ds(0, CH)], w_v.at[b, s],
                              si[b]).wait()

    @pl.loop(0, CH)
    def _(i):
      for d in range(d_feat // 16):
        rows_v[i, pl.ds(d * 16, 16)] = jnp.zeros((16,), jnp.float32)

    @pl.loop(0, CHUNKS_PER_TILE)
    def _(j):
      pltpu.sync_copy(rows_v.at[pl.ds(0, CH)],
                      acc_sh.at[pl.ds(sid * ROWS_PER_TILE + j * CH, CH)])

    plsc.subcore_barrier()

    start_idx(0, 0)
    start_idx(1, 1)

    @pl.loop(0, NCH // 2)
    def _(j):
      for b in (0, 1):
        k = 2 * j + b
        wait_idx(b)

        hg = [pltpu.async_copy(src.at[row_v.at[b, s]],
                               rows_v.at[pl.ds(s * CH, CH)], sg[s])
              for s in range(sub)]
        hs = []
        for s in range(sub):
          hg[s].wait()

          @pl.loop(0, CH)
          def _(i):
            w = plsc.load_gather(w_v.at[b, s], [jnp.full((16,), i, jnp.int32)])
            for d in range(d_feat // 16):
              sl = (s * CH + i, pl.ds(d * 16, 16))
              rows_v[sl] = rows_v[sl] * w

          hs.append(pltpu.async_copy(rows_v.at[pl.ds(s * CH, CH)],
                                     acc_sh.at[col_v.at[b, s]], ss[s],
                                     add=True))
        for h in hs:
          h.wait()

        nk = jnp.where(k + 2 >= NCH, k + 2 - NCH, k + 2)
        start_idx(nk, b)

    wait_idx(0)
    wait_idx(1)

    plsc.subcore_barrier()

    @pl.loop(0, CHUNKS_PER_TILE)
    def _(j):
      start = sid * ROWS_PER_TILE + j * CH
      pltpu.sync_copy(acc_sh.at[pl.ds(start, CH)],
                      out_hbm.at[cid, pl.ds(start, CH)])

  return k


_DEG_SUB = 4


@functools.partial(
    pl.kernel,
    out_type=jax.ShapeDtypeStruct((2, NP), jnp.float32),
    mesh=_MESH,
    compiler_params=_SC_PARAMS,
    scratch_types=[
        pltpu.VMEM((2, _DEG_SUB, CH), jnp.int32),
        pltpu.VMEM((2, _DEG_SUB, CH), jnp.float32),
        pltpu.VMEM_SHARED((NP,), jnp.float32),
        pltpu.SemaphoreType.DMA,
        pltpu.SemaphoreType.DMA,
    ],
)
def _degree_kernel(col_hbm, w_hbm, out_hbm, col_v, w_v, deg_sh, si0, si1):
  cid = lax.axis_index("c")
  sid = lax.axis_index("s")
  wid = cid * 16 + sid
  si = (si0, si1)
  chunk = _DEG_SUB * CH
  nch = EPW // chunk

  def start_idx(k, b):
    base = wid * EPW + k * chunk
    for s in range(_DEG_SUB):
      pltpu.async_copy(col_hbm.at[pl.ds(base + s * CH, CH)], col_v.at[b, s],
                       si[b])
      pltpu.async_copy(w_hbm.at[pl.ds(base + s * CH, CH)], w_v.at[b, s],
                       si[b])

  def wait_idx(b):
    for s in range(_DEG_SUB):
      pltpu.make_async_copy(col_hbm.at[pl.ds(0, CH)], col_v.at[b, s],
                            si[b]).wait()
      pltpu.make_async_copy(w_hbm.at[pl.ds(0, CH)], w_v.at[b, s],
                            si[b]).wait()

  @pl.loop(0, CH // 16)
  def _(g):
    w_v[0, 0, pl.ds(g * 16, 16)] = jnp.zeros((16,), jnp.float32)

  @pl.loop(0, CHUNKS_PER_TILE)
  def _(j):
    pltpu.sync_copy(w_v.at[0, 0],
                    deg_sh.at[pl.ds(sid * ROWS_PER_TILE + j * CH, CH)])

  plsc.subcore_barrier()
  start_idx(0, 0)
  start_idx(1, 1)

  @pl.loop(0, nch // 2)
  def _(j):
    for b in (0, 1):
      k = 2 * j + b
      wait_idx(b)
      for s in range(_DEG_SUB):
        pltpu.sync_copy(w_v.at[b, s], deg_sh.at[col_v.at[b, s]], add=True)
      nk = jnp.where(k + 2 >= nch, k + 2 - nch, k + 2)
      start_idx(nk, b)

  wait_idx(0)
  wait_idx(1)
  plsc.subcore_barrier()

  @pl.loop(0, CHUNKS_PER_TILE)
  def _(j):
    start = sid * ROWS_PER_TILE + j * CH
    pltpu.sync_copy(deg_sh.at[pl.ds(start, CH)], out_hbm.at[cid, pl.ds(start, CH)])


_GAT_SUB = 2
_GAT_CHUNK = _GAT_SUB * CH


@functools.partial(
    pl.kernel,
    out_type=[
        jax.ShapeDtypeStruct((2, NP), jnp.float32),
        jax.ShapeDtypeStruct((2, NP, 64), jnp.float32),
    ],
    mesh=_MESH,
    compiler_params=_SC_PARAMS_LINEAR,
    scratch_types=[
        pltpu.VMEM((2, _GAT_SUB, CH), jnp.int32),
        pltpu.VMEM((2, _GAT_SUB, CH), jnp.int32),
        pltpu.VMEM((_GAT_CHUNK,), jnp.float32),
        pltpu.VMEM((_GAT_CHUNK, 2), jnp.float32),
        pltpu.VMEM((_GAT_CHUNK,), jnp.float32),
        pltpu.VMEM((_GAT_CHUNK, 64), jnp.float32),
        pltpu.VMEM_SHARED((NP,), jnp.float32),
        pltpu.VMEM_SHARED((NP, 64), jnp.float32),
        pltpu.VMEM_SHARED((NP, 64), jnp.float32),
        pltpu.VMEM_SHARED((NP,), jnp.float32),
        pltpu.VMEM_SHARED((NP, 2), jnp.float32),
        pltpu.SemaphoreType.DMA,
        pltpu.SemaphoreType.DMA,
    ] + [pltpu.SemaphoreType.DMA] * (5 * _GAT_SUB),
)
def _gat_edge_kernel(h3_hbm, asrc_hbm, pq_hbm, row_hbm, col_hbm,
                     s_out, acc_out, row_v, col_v, as_v, pq_v, ex_v, rows_v,
                     s_sh, acc_sh, h3_sh, as_sh, pq_sh, si0, si1, *sems):
  cid = lax.axis_index("c")
  sid = lax.axis_index("s")
  wid = cid * 16 + sid
  si = (si0, si1)
  nch = EPW // _GAT_CHUNK

  def start_idx(k, b):
    base = wid * EPW + k * _GAT_CHUNK
    for s in range(_GAT_SUB):
      pltpu.async_copy(row_hbm.at[pl.ds(base + s * CH, CH)], row_v.at[b, s],
                       si[b])
      pltpu.async_copy(col_hbm.at[pl.ds(base + s * CH, CH)], col_v.at[b, s],
                       si[b])

  def wait_idx(b):
    for s in range(_GAT_SUB):
      pltpu.make_async_copy(row_hbm.at[pl.ds(0, CH)], row_v.at[b, s],
                            si[b]).wait()
      pltpu.make_async_copy(col_hbm.at[pl.ds(0, CH)], col_v.at[b, s],
                            si[b]).wait()

  @pl.loop(0, CH)
  def _(i):
    for d in range(4):
      rows_v[i, pl.ds(d * 16, 16)] = jnp.zeros((16,), jnp.float32)

  @pl.loop(0, CH // 16)
  def _(g):
    ex_v[pl.ds(g * 16, 16)] = jnp.zeros((16,), jnp.float32)

  @pl.loop(0, CHUNKS_PER_TILE)
  def _(j):
    start = sid * ROWS_PER_TILE + j * CH
    pltpu.sync_copy(rows_v.at[pl.ds(0, CH)], acc_sh.at[pl.ds(start, CH)])
    pltpu.sync_copy(ex_v.at[pl.ds(0, CH)], s_sh.at[pl.ds(start, CH)])

  tile = pl.ds(sid * ROWS_PER_TILE, ROWS_PER_TILE)
  pltpu.sync_copy(h3_hbm.at[tile], h3_sh.at[tile])
  pltpu.sync_copy(asrc_hbm.at[tile], as_sh.at[tile])
  pltpu.sync_copy(pq_hbm.at[tile], pq_sh.at[tile])

  plsc.subcore_barrier()
  start_idx(0, 0)
  start_idx(1, 1)

  @pl.loop(0, nch // 2)
  def _(j):
    for b in (0, 1):
      k = 2 * j + b
      wait_idx(b)

      sa = sems[:_GAT_SUB]
      sp = sems[_GAT_SUB:2 * _GAT_SUB]
      sh = sems[2 * _GAT_SUB:3 * _GAT_SUB]
      se = sems[3 * _GAT_SUB:4 * _GAT_SUB]
      sr = sems[4 * _GAT_SUB:5 * _GAT_SUB]
      ha = [pltpu.async_copy(as_sh.at[row_v.at[b, s]],
                             as_v.at[pl.ds(s * CH, CH)], sa[s])
            for s in range(_GAT_SUB)]
      hp = [pltpu.async_copy(pq_sh.at[col_v.at[b, s]],
                             pq_v.at[pl.ds(s * CH, CH)], sp[s])
            for s in range(_GAT_SUB)]
      hh = [pltpu.async_copy(h3_sh.at[row_v.at[b, s]],
                             rows_v.at[pl.ds(s * CH, CH)], sh[s])
            for s in range(_GAT_SUB)]
      hw = []
      for s in range(_GAT_SUB):
        ha[s].wait()
        hp[s].wait()

        @pl.loop(0, CH // 16)
        def _(g):
          lane = lax.iota(jnp.int32, 16) + (s * CH + g * 16)
          ad = plsc.load_gather(pq_v, [lane, jnp.zeros((16,), jnp.int32)])
          off = plsc.load_gather(pq_v, [lane, jnp.ones((16,), jnp.int32)])
          sl = pl.ds(s * CH + g * 16, 16)
          z = as_v[sl] + ad
          e = jnp.where(z > 0.0, z, 0.2 * z)
          ex_v[sl] = jnp.exp(e - off)

        hw.append(pltpu.async_copy(ex_v.at[pl.ds(s * CH, CH)],
                                   s_sh.at[col_v.at[b, s]], se[s], add=True))
      for s in range(_GAT_SUB):
        hh[s].wait()

        @pl.loop(0, CH)
        def _(i):
          ii = s * CH + i
          w = plsc.load_gather(ex_v, [jnp.full((16,), ii, jnp.int32)])
          for d in range(4):
            sl = (ii, pl.ds(d * 16, 16))
            rows_v[sl] = rows_v[sl] * w

        hw.append(pltpu.async_copy(rows_v.at[pl.ds(s * CH, CH)],
                                   acc_sh.at[col_v.at[b, s]], sr[s], add=True))
      for h in hw:
        h.wait()

      nk = jnp.where(k + 2 >= nch, k + 2 - nch, k + 2)
      start_idx(nk, b)

  wait_idx(0)
  wait_idx(1)
  plsc.subcore_barrier()

  @pl.loop(0, CHUNKS_PER_TILE)
  def _(j):
    start = sid * ROWS_PER_TILE + j * CH
    pltpu.sync_copy(s_sh.at[pl.ds(start, CH)], s_out.at[cid, pl.ds(start, CH)])
    pltpu.sync_copy(acc_sh.at[pl.ds(start, CH)],
                    acc_out.at[cid, pl.ds(start, CH)])


BR = 2000
GRID = N_NODES // BR


def _rb(d):
  return pl.BlockSpec((BR, d), lambda i: (i, 0))


def _full(s0, s1):
  return pl.BlockSpec((s0, s1), lambda i: (0, 0))


def _part(core, d):
  return pl.BlockSpec((1, BR, d), lambda i, c=core: (c, i, 0))


def _row_call(body, in_specs, out_shape, out_specs):
  return pl.pallas_call(body, grid=(GRID,), in_specs=in_specs,
                        out_shape=out_shape, out_specs=out_specs)


_DOT = functools.partial(jnp.dot, preferred_element_type=jnp.float32,
                         precision=lax.Precision.HIGHEST)


def _mm_kernel(x_ref, w_ref, o_ref):
  o_ref[...] = _DOT(x_ref[...], w_ref[...])


def _scale_kernel(hp_ref, d0_ref, d1_ref, g_ref, dinv_ref):
  deg = d0_ref[0] + d1_ref[0] + 1.0
  dinv = lax.rsqrt(deg)
  dinv_ref[...] = dinv
  g_ref[...] = hp_ref[...] * dinv


def _combine_kernel(a0_ref, a1_ref, hp_ref, dinv_ref, b_ref, w_ref,
                    hnext_ref, gnext_ref):
  dinv = dinv_ref[...]
  out = dinv * (a0_ref[0] + a1_ref[0]) + dinv * dinv * hp_ref[...] \
      + b_ref[...]
  h = out * (1.0 / (1.0 + jnp.exp(-out)))
  hp = _DOT(h, w_ref[...])
  hnext_ref[...] = hp
  gnext_ref[...] = hp * dinv


def _gat_mm_kernel(a0_ref, a1_ref, hp_ref, dinv_ref, b_ref, w_ref,
                   atts_ref, attd_ref, h3_ref, asrc_ref, adst_ref):
  dinv = dinv_ref[...]
  out = dinv * (a0_ref[0] + a1_ref[0]) + dinv * dinv * hp_ref[...] \
      + b_ref[...]
  h2 = out * (1.0 / (1.0 + jnp.exp(-out)))
  h3 = _DOT(h2, w_ref[...])
  h3_ref[...] = h3
  asrc_ref[...] = _DOT(h3, atts_ref[...])
  adst_ref[...] = _DOT(h3, attd_ref[...])


def _att_prep_kernel(asrc_ref, adst_ref, pq_ref, exs_ref):
  asrc = asrc_ref[...]
  adst = adst_ref[...]
  amax = jnp.max(asrc)
  es = asrc + adst
  e_self = jnp.where(es > 0.0, es, 0.2 * es)
  off = jnp.maximum(e_self, adst + amax)
  pq_ref[...] = jnp.concatenate([adst, off], axis=1)
  exs_ref[...] = jnp.exp(e_self - off)


def _final_kernel(a0_ref, a1_ref, s0_ref, s1_ref, exs_ref, h3_ref, b_ref,
                  o_ref):
  s = s0_ref[0] + s1_ref[0] + exs_ref[...]
  num = a0_ref[0] + a1_ref[0] + exs_ref[...] * h3_ref[...]
  o3 = num / s + b_ref[...]
  m = jnp.max(o3, axis=1, keepdims=True)
  e = jnp.exp(o3 - m)
  o_ref[...] = e / jnp.sum(e, axis=1, keepdims=True)


def kernel(x, edge_index, edge_weight, W1, b1, W2, b2, W3, att_src, att_dst,
           b3):
  n = N_NODES
  row, col = edge_index[0], edge_index[1]

  pad = EPAD - N_EDGES
  pad_idx = (jnp.arange(pad, dtype=jnp.int32) * 8) % n
  row_p = jnp.concatenate([row, pad_idx])
  col_p = jnp.concatenate([col, pad_idx])
  ew_p = jnp.concatenate([edge_weight, jnp.zeros((pad,), jnp.float32)])
  row_q = jnp.concatenate(
      [row, n + (jnp.arange(pad, dtype=jnp.int32) % 16)])

  deg_parts = _degree_kernel(col_p, ew_p)
  h1p = _row_call(_mm_kernel, [_rb(128), _full(128, 128)],
                  jax.ShapeDtypeStruct((n, 128), jnp.float32),
                  _rb(128))(x, W1)

  dp = deg_parts.reshape(2, NP, 1)
  g1, dinv = _row_call(
      _scale_kernel, [_rb(128), _part(0, 1), _part(1, 1)],
      (jax.ShapeDtypeStruct((n, 128), jnp.float32),
       jax.ShapeDtypeStruct((n, 1), jnp.float32)),
      (_rb(128), _rb(1)))(h1p, dp, dp)

  acc1 = _edge_accumulate(128, 2, False)(g1, row_p, col_p, ew_p)
  h2p, g2 = _row_call(
      _combine_kernel,
      [_part(0, 128), _part(1, 128), _rb(128), _rb(1), _full(1, 128),
       _full(128, 64)],
      (jax.ShapeDtypeStruct((n, 64), jnp.float32),
       jax.ShapeDtypeStruct((n, 64), jnp.float32)),
      (_rb(64), _rb(64)))(
          acc1, acc1, h1p, dinv, b1.reshape(1, 128), W2)

  g2x = jnp.concatenate([g2, jnp.zeros((NP - n, 64), jnp.float32)])
  acc2 = _edge_accumulate(64, 2, True)(g2x, row_p, col_p, ew_p)
  h3, asrc, adst = _row_call(
      _gat_mm_kernel,
      [_part(0, 64), _part(1, 64), _rb(64), _rb(1), _full(1, 64),
       _full(64, 64), _full(64, 1), _full(64, 1)],
      (jax.ShapeDtypeStruct((n, 64), jnp.float32),
       jax.ShapeDtypeStruct((n, 1), jnp.float32),
       jax.ShapeDtypeStruct((n, 1), jnp.float32)),
      (_rb(64), _rb(1), _rb(1)))(
          acc2, acc2, h2p, dinv, b2.reshape(1, 64), W3,
          att_src.reshape(64, 1), att_dst.reshape(64, 1))

  pq, exs = pl.pallas_call(
      _att_prep_kernel,
      out_shape=(jax.ShapeDtypeStruct((n, 2), jnp.float32),
                 jax.ShapeDtypeStruct((n, 1), jnp.float32)))(asrc, adst)

  asx = jnp.concatenate(
      [asrc.reshape(n), jnp.full((NP - n,), -1e30, jnp.float32)])
  h3x = jnp.concatenate([h3, jnp.zeros((NP - n, 64), jnp.float32)])
  pqx = jnp.concatenate([pq, jnp.zeros((NP - n, 2), jnp.float32)])
  s_parts, acc3 = _gat_edge_kernel(h3x, asx, pqx, row_q, col_p)

  sp = s_parts.reshape(2, NP, 1)
  out = _row_call(
      _final_kernel,
      [_part(0, 64), _part(1, 64), _part(0, 1), _part(1, 1), _rb(1), _rb(64),
       _full(1, 64)],
      jax.ShapeDtypeStruct((n, 64), jnp.float32),
      _rb(64))(
          acc3, acc3, sp, sp, exs, h3, b3.reshape(1, 64))
  return out


# speedup vs baseline: 34.4023x; 1.1593x over previous
import dataclasses
import functools

import jax
import jax.numpy as jnp
from jax import lax
from jax.experimental import pallas as pl
from jax.experimental.pallas import tpu as pltpu
from jax.experimental.pallas import tpu_sc as plsc

N_NODES = 10000
N_EDGES = 320000
NP = 10240
N_WORKERS = 32
CH = 128
EPW = 10240
EPAD = EPW * N_WORKERS
ROWS_PER_TILE = NP // 16
CHUNKS_PER_TILE = ROWS_PER_TILE // CH

_MESH = plsc.VectorSubcoreMesh(core_axis_name="c", subcore_axis_name="s")

_SC_PARAMS = pltpu.CompilerParams()
if "needs_layout_passes" in pltpu.CompilerParams.__dataclass_fields__:
  _SC_PARAMS = dataclasses.replace(_SC_PARAMS, needs_layout_passes=False)
_SC_PARAMS_LINEAR = dataclasses.replace(_SC_PARAMS, use_tc_tiling_on_sc=False)


def _edge_accumulate(d_feat, sub, staged):
  CHUNK = CH * sub
  NCH = EPW // CHUNK

  scratch = [
      pltpu.VMEM((2, sub, CH), jnp.int32),
      pltpu.VMEM((2, sub, CH), jnp.int32),
      pltpu.VMEM((2, sub, CH), jnp.float32),
      pltpu.VMEM((CHUNK, d_feat), jnp.float32),
      pltpu.VMEM_SHARED((NP, d_feat), jnp.float32),
      pltpu.SemaphoreType.DMA,
      pltpu.SemaphoreType.DMA,
  ]
  scratch += [pltpu.SemaphoreType.DMA] * (2 * sub)
  if staged:
    scratch.append(pltpu.VMEM_SHARED((NP, d_feat), jnp.float32))

  @functools.partial(
      pl.kernel,
      out_type=jax.ShapeDtypeStruct((2, NP, d_feat), jnp.float32),
      mesh=_MESH,
      compiler_params=_SC_PARAMS if d_feat == 128 else _SC_PARAMS_LINEAR,
      scratch_types=scratch,
  )
  def k(src_hbm, row_hbm, col_hbm, w_hbm, out_hbm, row_v, col_v, w_v,
        rows_v, acc_sh, si0, si1, *rest):
    cid = lax.axis_index("c")
    sid = lax.axis_index("s")
    wid = cid * 16 + sid
    si = (si0, si1)
    sg = rest[:sub]
    ss = rest[sub:2 * sub]
    src = rest[2 * sub] if staged else src_hbm
    if staged:
      pltpu.sync_copy(src_hbm.at[pl.ds(sid * ROWS_PER_TILE, ROWS_PER_TILE)],
                      src.at[pl.ds(sid * ROWS_PER_TILE, ROWS_PER_TILE)])

    def start_idx(chunk, b):
      base = wid * EPW + chunk * CHUNK
      for s in range(sub):
        pltpu.async_copy(row_hbm.at[pl.ds(base + s * CH, CH)],
                         row_v.at[b, s], si[b])
        pltpu.async_copy(col_hbm.at[pl.ds(base + s * CH, CH)],
                         col_v.at[b, s], si[b])
        pltpu.async_copy(w_hbm.at[pl.ds(base + s * CH, CH)],
                         w_v.at[b, s], si[b])

    def wait_idx(b):
      for s in range(sub):
        pltpu.make_async_copy(row_hbm.at[pl.ds(0, CH)], row_v.at[b, s],
                              si[b]).wait()
        pltpu.make_async_copy(col_hbm.at[pl.ds(0, CH)], col_v.at[b, s],
                              si[b]).wait()
        pltpu.make_async_copy(w_hbm.at[pl.ds(0, CH)], w_v.at[b, s],
                              si[b]).wait()

    @pl.loop(0, CH)
    def _(i):
      for d in range(d_feat // 16):
        rows_v[i, pl.ds(d * 16, 16)] = jnp.zeros((16,), jnp.float32)

    @pl.loop(0, CHUNKS_PER_TILE)
    def _(j):
      pltpu.sync_copy(rows_v.at[pl.ds(0, CH)],
                      acc_sh.at[pl.ds(sid * ROWS_PER_TILE + j * CH, CH)])

    plsc.subcore_barrier()

    start_idx(0, 0)
    start_idx(1, 1)

    @pl.loop(0, NCH // 2)
    def _(j):
      for b in (0, 1):
        k = 2 * j + b
        wait_idx(b)

        hg = [pltpu.async_copy(src.at[row_v.at[b, s]],
                               rows_v.at[pl.ds(s * CH, CH)], sg[s])
              for s in range(sub)]
        hs = []
        for s in range(sub):
          hg[s].wait()

          @plsc.parallel_loop(0, CH, unroll=4)
          def _(i):
            w = plsc.load_gather(w_v.at[b, s], [jnp.full((16,), i, jnp.int32)])
            for d in range(d_feat // 16):
              sl = (s * CH + i, pl.ds(d * 16, 16))
              rows_v[sl] = rows_v[sl] * w

          hs.append(pltpu.async_copy(rows_v.at[pl.ds(s * CH, CH)],
                                     acc_sh.at[col_v.at[b, s]], ss[s],
                                     add=True))
        for h in hs:
          h.wait()

        nk = jnp.where(k + 2 >= NCH, k + 2 - NCH, k + 2)
        start_idx(nk, b)

    wait_idx(0)
    wait_idx(1)

    plsc.subcore_barrier()

    @pl.loop(0, CHUNKS_PER_TILE)
    def _(j):
      start = sid * ROWS_PER_TILE + j * CH
      pltpu.sync_copy(acc_sh.at[pl.ds(start, CH)],
                      out_hbm.at[cid, pl.ds(start, CH)])

  return k


_DEG_SUB = 4


@functools.partial(
    pl.kernel,
    out_type=jax.ShapeDtypeStruct((2, NP), jnp.float32),
    mesh=_MESH,
    compiler_params=_SC_PARAMS,
    scratch_types=[
        pltpu.VMEM((2, _DEG_SUB, CH), jnp.int32),
        pltpu.VMEM((2, _DEG_SUB, CH), jnp.float32),
        pltpu.VMEM_SHARED((NP,), jnp.float32),
        pltpu.SemaphoreType.DMA,
        pltpu.SemaphoreType.DMA,
    ],
)
def _degree_kernel(col_hbm, w_hbm, out_hbm, col_v, w_v, deg_sh, si0, si1):
  cid = lax.axis_index("c")
  sid = lax.axis_index("s")
  wid = cid * 16 + sid
  si = (si0, si1)
  chunk = _DEG_SUB * CH
  nch = EPW // chunk

  def start_idx(k, b):
    base = wid * EPW + k * chunk
    for s in range(_DEG_SUB):
      pltpu.async_copy(col_hbm.at[pl.ds(base + s * CH, CH)], col_v.at[b, s],
                       si[b])
      pltpu.async_copy(w_hbm.at[pl.ds(base + s * CH, CH)], w_v.at[b, s],
                       si[b])

  def wait_idx(b):
    for s in range(_DEG_SUB):
      pltpu.make_async_copy(col_hbm.at[pl.ds(0, CH)], col_v.at[b, s],
                            si[b]).wait()
      pltpu.make_async_copy(w_hbm.at[pl.ds(0, CH)], w_v.at[b, s],
                            si[b]).wait()

  @pl.loop(0, CH // 16)
  def _(g):
    w_v[0, 0, pl.ds(g * 16, 16)] = jnp.zeros((16,), jnp.float32)

  @pl.loop(0, CHUNKS_PER_TILE)
  def _(j):
    pltpu.sync_copy(w_v.at[0, 0],
                    deg_sh.at[pl.ds(sid * ROWS_PER_TILE + j * CH, CH)])

  plsc.subcore_barrier()
  start_idx(0, 0)
  start_idx(1, 1)

  @pl.loop(0, nch // 2)
  def _(j):
    for b in (0, 1):
      k = 2 * j + b
      wait_idx(b)
      for s in range(_DEG_SUB):
        pltpu.sync_copy(w_v.at[b, s], deg_sh.at[col_v.at[b, s]], add=True)
      nk = jnp.where(k + 2 >= nch, k + 2 - nch, k + 2)
      start_idx(nk, b)

  wait_idx(0)
  wait_idx(1)
  plsc.subcore_barrier()

  @pl.loop(0, CHUNKS_PER_TILE)
  def _(j):
    start = sid * ROWS_PER_TILE + j * CH
    pltpu.sync_copy(deg_sh.at[pl.ds(start, CH)], out_hbm.at[cid, pl.ds(start, CH)])


_GAT_SUB = 2
_GAT_CHUNK = _GAT_SUB * CH


@functools.partial(
    pl.kernel,
    out_type=[
        jax.ShapeDtypeStruct((2, NP), jnp.float32),
        jax.ShapeDtypeStruct((2, NP, 64), jnp.float32),
    ],
    mesh=_MESH,
    compiler_params=_SC_PARAMS_LINEAR,
    scratch_types=[
        pltpu.VMEM((2, _GAT_SUB, CH), jnp.int32),
        pltpu.VMEM((2, _GAT_SUB, CH), jnp.int32),
        pltpu.VMEM((_GAT_CHUNK,), jnp.float32),
        pltpu.VMEM((_GAT_CHUNK, 2), jnp.float32),
        pltpu.VMEM((_GAT_CHUNK,), jnp.float32),
        pltpu.VMEM((_GAT_CHUNK, 64), jnp.float32),
        pltpu.VMEM_SHARED((NP,), jnp.float32),
        pltpu.VMEM_SHARED((NP, 64), jnp.float32),
        pltpu.VMEM_SHARED((NP, 64), jnp.float32),
        pltpu.VMEM_SHARED((NP,), jnp.float32),
        pltpu.VMEM_SHARED((NP, 2), jnp.float32),
        pltpu.SemaphoreType.DMA,
        pltpu.SemaphoreType.DMA,
    ] + [pltpu.SemaphoreType.DMA] * (5 * _GAT_SUB),
)
def _gat_edge_kernel(h3_hbm, asrc_hbm, pq_hbm, row_hbm, col_hbm,
                     s_out, acc_out, row_v, col_v, as_v, pq_v, ex_v, rows_v,
                     s_sh, acc_sh, h3_sh, as_sh, pq_sh, si0, si1, *sems):
  cid = lax.axis_index("c")
  sid = lax.axis_index("s")
  wid = cid * 16 + sid
  si = (si0, si1)
  nch = EPW // _GAT_CHUNK

  def start_idx(k, b):
    base = wid * EPW + k * _GAT_CHUNK
    for s in range(_GAT_SUB):
      pltpu.async_copy(row_hbm.at[pl.ds(base + s * CH, CH)], row_v.at[b, s],
                       si[b])
      pltpu.async_copy(col_hbm.at[pl.ds(base + s * CH, CH)], col_v.at[b, s],
                       si[b])

  def wait_idx(b):
    for s in range(_GAT_SUB):
      pltpu.make_async_copy(row_hbm.at[pl.ds(0, CH)], row_v.at[b, s],
                            si[b]).wait()
      pltpu.make_async_copy(col_hbm.at[pl.ds(0, CH)], col_v.at[b, s],
                            si[b]).wait()

  @pl.loop(0, CH)
  def _(i):
    for d in range(4):
      rows_v[i, pl.ds(d * 16, 16)] = jnp.zeros((16,), jnp.float32)

  @pl.loop(0, CH // 16)
  def _(g):
    ex_v[pl.ds(g * 16, 16)] = jnp.zeros((16,), jnp.float32)

  @pl.loop(0, CHUNKS_PER_TILE)
  def _(j):
    start = sid * ROWS_PER_TILE + j * CH
    pltpu.sync_copy(rows_v.at[pl.ds(0, CH)], acc_sh.at[pl.ds(start, CH)])
    pltpu.sync_copy(ex_v.at[pl.ds(0, CH)], s_sh.at[pl.ds(start, CH)])

  tile = pl.ds(sid * ROWS_PER_TILE, ROWS_PER_TILE)
  pltpu.sync_copy(h3_hbm.at[tile], h3_sh.at[tile])
  pltpu.sync_copy(asrc_hbm.at[tile], as_sh.at[tile])
  pltpu.sync_copy(pq_hbm.at[tile], pq_sh.at[tile])

  plsc.subcore_barrier()
  start_idx(0, 0)
  start_idx(1, 1)

  @pl.loop(0, nch // 2)
  def _(j):
    for b in (0, 1):
      k = 2 * j + b
      wait_idx(b)

      sa = sems[:_GAT_SUB]
      sp = sems[_GAT_SUB:2 * _GAT_SUB]
      sh = sems[2 * _GAT_SUB:3 * _GAT_SUB]
      se = sems[3 * _GAT_SUB:4 * _GAT_SUB]
      sr = sems[4 * _GAT_SUB:5 * _GAT_SUB]
      ha = [pltpu.async_copy(as_sh.at[row_v.at[b, s]],
                             as_v.at[pl.ds(s * CH, CH)], sa[s])
            for s in range(_GAT_SUB)]
      hp = [pltpu.async_copy(pq_sh.at[col_v.at[b, s]],
                             pq_v.at[pl.ds(s * CH, CH)], sp[s])
            for s in range(_GAT_SUB)]
      hh = [pltpu.async_copy(h3_sh.at[row_v.at[b, s]],
                             rows_v.at[pl.ds(s * CH, CH)], sh[s])
            for s in range(_GAT_SUB)]
      hw = []
      for s in range(_GAT_SUB):
        ha[s].wait()
        hp[s].wait()

        @plsc.parallel_loop(0, CH // 16, unroll=2)
        def _(g):
          lane = lax.iota(jnp.int32, 16) + (s * CH + g * 16)
          ad = plsc.load_gather(pq_v, [lane, jnp.zeros((16,), jnp.int32)])
          off = plsc.load_gather(pq_v, [lane, jnp.ones((16,), jnp.int32)])
          sl = pl.ds(s * CH + g * 16, 16)
          z = as_v[sl] + ad
          e = jnp.where(z > 0.0, z, 0.2 * z)
          ex_v[sl] = jnp.exp(e - off)

        hw.append(pltpu.async_copy(ex_v.at[pl.ds(s * CH, CH)],
                                   s_sh.at[col_v.at[b, s]], se[s], add=True))
      for s in range(_GAT_SUB):
        hh[s].wait()

        @plsc.parallel_loop(0, CH, unroll=4)
        def _(i):
          ii = s * CH + i
          w = plsc.load_gather(ex_v, [jnp.full((16,), ii, jnp.int32)])
          for d in range(4):
            sl = (ii, pl.ds(d * 16, 16))
            rows_v[sl] = rows_v[sl] * w

        hw.append(pltpu.async_copy(rows_v.at[pl.ds(s * CH, CH)],
                                   acc_sh.at[col_v.at[b, s]], sr[s], add=True))
      for h in hw:
        h.wait()

      nk = jnp.where(k + 2 >= nch, k + 2 - nch, k + 2)
      start_idx(nk, b)

  wait_idx(0)
  wait_idx(1)
  plsc.subcore_barrier()

  @pl.loop(0, CHUNKS_PER_TILE)
  def _(j):
    start = sid * ROWS_PER_TILE + j * CH
    pltpu.sync_copy(s_sh.at[pl.ds(start, CH)], s_out.at[cid, pl.ds(start, CH)])
    pltpu.sync_copy(acc_sh.at[pl.ds(start, CH)],
                    acc_out.at[cid, pl.ds(start, CH)])


BR = 2000
GRID = N_NODES // BR


def _rb(d):
  return pl.BlockSpec((BR, d), lambda i: (i, 0))


def _full(s0, s1):
  return pl.BlockSpec((s0, s1), lambda i: (0, 0))


def _part(core, d):
  return pl.BlockSpec((1, BR, d), lambda i, c=core: (c, i, 0))


def _row_call(body, in_specs, out_shape, out_specs):
  return pl.pallas_call(body, grid=(GRID,), in_specs=in_specs,
                        out_shape=out_shape, out_specs=out_specs)


_DOT = functools.partial(jnp.dot, preferred_element_type=jnp.float32,
                         precision=lax.Precision.HIGHEST)


def _mm_kernel(x_ref, w_ref, o_ref):
  o_ref[...] = _DOT(x_ref[...], w_ref[...])


def _scale_kernel(hp_ref, d0_ref, d1_ref, g_ref, dinv_ref):
  deg = d0_ref[0] + d1_ref[0] + 1.0
  dinv = lax.rsqrt(deg)
  dinv_ref[...] = dinv
  g_ref[...] = hp_ref[...] * dinv


def _combine_kernel(a0_ref, a1_ref, hp_ref, dinv_ref, b_ref, w_ref,
                    hnext_ref, gnext_ref):
  dinv = dinv_ref[...]
  out = dinv * (a0_ref[0] + a1_ref[0]) + dinv * dinv * hp_ref[...] \
      + b_ref[...]
  h = out * (1.0 / (1.0 + jnp.exp(-out)))
  hp = _DOT(h, w_ref[...])
  hnext_ref[...] = hp
  gnext_ref[...] = hp * dinv


def _gat_mm_kernel(a0_ref, a1_ref, hp_ref, dinv_ref, b_ref, w_ref,
                   atts_ref, attd_ref, h3_ref, asrc_ref, adst_ref):
  dinv = dinv_ref[...]
  out = dinv * (a0_ref[0] + a1_ref[0]) + dinv * dinv * hp_ref[...] \
      + b_ref[...]
  h2 = out * (1.0 / (1.0 + jnp.exp(-out)))
  h3 = _DOT(h2, w_ref[...])
  h3_ref[...] = h3
  asrc_ref[...] = _DOT(h3, atts_ref[...])
  adst_ref[...] = _DOT(h3, attd_ref[...])


def _att_prep_kernel(asrc_ref, adst_ref, pq_ref, exs_ref):
  asrc = asrc_ref[...]
  adst = adst_ref[...]
  amax = jnp.max(asrc)
  es = asrc + adst
  e_self = jnp.where(es > 0.0, es, 0.2 * es)
  off = jnp.maximum(e_self, adst + amax)
  pq_ref[...] = jnp.concatenate([adst, off], axis=1)
  exs_ref[...] = jnp.exp(e_self - off)


def _final_kernel(a0_ref, a1_ref, s0_ref, s1_ref, exs_ref, h3_ref, b_ref,
                  o_ref):
  s = s0_ref[0] + s1_ref[0] + exs_ref[...]
  num = a0_ref[0] + a1_ref[0] + exs_ref[...] * h3_ref[...]
  o3 = num / s + b_ref[...]
  m = jnp.max(o3, axis=1, keepdims=True)
  e = jnp.exp(o3 - m)
  o_ref[...] = e / jnp.sum(e, axis=1, keepdims=True)


def kernel(x, edge_index, edge_weight, W1, b1, W2, b2, W3, att_src, att_dst,
           b3):
  n = N_NODES
  row, col = edge_index[0], edge_index[1]

  pad = EPAD - N_EDGES
  pad_idx = (jnp.arange(pad, dtype=jnp.int32) * 8) % n
  row_p = jnp.concatenate([row, pad_idx])
  col_p = jnp.concatenate([col, pad_idx])
  ew_p = jnp.concatenate([edge_weight, jnp.zeros((pad,), jnp.float32)])
  row_q = jnp.concatenate(
      [row, n + (jnp.arange(pad, dtype=jnp.int32) % 16)])

  deg_parts = _degree_kernel(col_p, ew_p)
  h1p = _row_call(_mm_kernel, [_rb(128), _full(128, 128)],
                  jax.ShapeDtypeStruct((n, 128), jnp.float32),
                  _rb(128))(x, W1)

  dp = deg_parts.reshape(2, NP, 1)
  g1, dinv = _row_call(
      _scale_kernel, [_rb(128), _part(0, 1), _part(1, 1)],
      (jax.ShapeDtypeStruct((n, 128), jnp.float32),
       jax.ShapeDtypeStruct((n, 1), jnp.float32)),
      (_rb(128), _rb(1)))(h1p, dp, dp)

  acc1 = _edge_accumulate(128, 2, False)(g1, row_p, col_p, ew_p)
  h2p, g2 = _row_call(
      _combine_kernel,
      [_part(0, 128), _part(1, 128), _rb(128), _rb(1), _full(1, 128),
       _full(128, 64)],
      (jax.ShapeDtypeStruct((n, 64), jnp.float32),
       jax.ShapeDtypeStruct((n, 64), jnp.float32)),
      (_rb(64), _rb(64)))(
          acc1, acc1, h1p, dinv, b1.reshape(1, 128), W2)

  g2x = jnp.concatenate([g2, jnp.zeros((NP - n, 64), jnp.float32)])
  acc2 = _edge_accumulate(64, 2, True)(g2x, row_p, col_p, ew_p)
  h3, asrc, adst = _row_call(
      _gat_mm_kernel,
      [_part(0, 64), _part(1, 64), _rb(64), _rb(1), _full(1, 64),
       _full(64, 64), _full(64, 1), _full(64, 1)],
      (jax.ShapeDtypeStruct((n, 64), jnp.float32),
       jax.ShapeDtypeStruct((n, 1), jnp.float32),
       jax.ShapeDtypeStruct((n, 1), jnp.float32)),
      (_rb(64), _rb(1), _rb(1)))(
          acc2, acc2, h2p, dinv, b2.reshape(1, 64), W3,
          att_src.reshape(64, 1), att_dst.reshape(64, 1))

  pq, exs = pl.pallas_call(
      _att_prep_kernel,
      out_shape=(jax.ShapeDtypeStruct((n, 2), jnp.float32),
                 jax.ShapeDtypeStruct((n, 1), jnp.float32)))(asrc, adst)

  asx = jnp.concatenate(
      [asrc.reshape(n), jnp.full((NP - n,), -1e30, jnp.float32)])
  h3x = jnp.concatenate([h3, jnp.zeros((NP - n, 64), jnp.float32)])
  pqx = jnp.concatenate([pq, jnp.zeros((NP - n, 2), jnp.float32)])
  s_parts, acc3 = _gat_edge_kernel(h3x, asx, pqx, row_q, col_p)

  sp = s_parts.reshape(2, NP, 1)
  out = _row_call(
      _final_kernel,
      [_part(0, 64), _part(1, 64), _part(0, 1), _part(1, 1), _rb(1), _rb(64),
       _full(1, 64)],
      jax.ShapeDtypeStruct((n, 64), jnp.float32),
      _rb(64))(
          acc3, acc3, sp, sp, exs, h3, b3.reshape(1, 64))
  return out


# speedup vs baseline: 36.7884x; 1.0694x over previous
import dataclasses
import functools

import jax
import jax.numpy as jnp
from jax import lax
from jax.experimental import pallas as pl
from jax.experimental.pallas import tpu as pltpu
from jax.experimental.pallas import tpu_sc as plsc

N_NODES = 10000
N_EDGES = 320000
NP = 10240
N_WORKERS = 32
CH = 128
EPW = 10240
EPAD = EPW * N_WORKERS
ROWS_PER_TILE = NP // 16
CHUNKS_PER_TILE = ROWS_PER_TILE // CH

_MESH = plsc.VectorSubcoreMesh(core_axis_name="c", subcore_axis_name="s")

_SC_PARAMS = pltpu.CompilerParams()
if "needs_layout_passes" in pltpu.CompilerParams.__dataclass_fields__:
  _SC_PARAMS = dataclasses.replace(_SC_PARAMS, needs_layout_passes=False)
_SC_PARAMS_LINEAR = dataclasses.replace(_SC_PARAMS, use_tc_tiling_on_sc=False)


def _edge_accumulate(d_feat, sub, staged):
  CHUNK = CH * sub
  NCH = EPW // CHUNK

  scratch = [
      pltpu.VMEM((2, sub, CH), jnp.int32),
      pltpu.VMEM((2, sub, CH), jnp.int32),
      pltpu.VMEM((2, sub, CH), jnp.float32),
      pltpu.VMEM((CHUNK, d_feat), jnp.float32),
      pltpu.VMEM_SHARED((NP, d_feat), jnp.float32),
      pltpu.SemaphoreType.DMA,
      pltpu.SemaphoreType.DMA,
  ]
  scratch += [pltpu.SemaphoreType.DMA] * (2 * sub)
  if staged:
    scratch.append(pltpu.VMEM_SHARED((NP, d_feat), jnp.float32))

  @functools.partial(
      pl.kernel,
      out_type=jax.ShapeDtypeStruct((2, NP, d_feat), jnp.float32),
      mesh=_MESH,
      compiler_params=_SC_PARAMS if d_feat == 128 else _SC_PARAMS_LINEAR,
      scratch_types=scratch,
  )
  def k(src_hbm, row_hbm, col_hbm, w_hbm, out_hbm, row_v, col_v, w_v,
        rows_v, acc_sh, si0, si1, *rest):
    cid = lax.axis_index("c")
    sid = lax.axis_index("s")
    wid = cid * 16 + sid
    si = (si0, si1)
    sg = rest[:sub]
    ss = rest[sub:2 * sub]
    src = rest[2 * sub] if staged else src_hbm
    if staged:
      pltpu.sync_copy(src_hbm.at[pl.ds(sid * ROWS_PER_TILE, ROWS_PER_TILE)],
                      src.at[pl.ds(sid * ROWS_PER_TILE, ROWS_PER_TILE)])

    def start_idx(chunk, b):
      base = wid * EPW + chunk * CHUNK
      for s in range(sub):
        pltpu.async_copy(row_hbm.at[pl.ds(base + s * CH, CH)],
                         row_v.at[b, s], si[b])
        pltpu.async_copy(col_hbm.at[pl.ds(base + s * CH, CH)],
                         col_v.at[b, s], si[b])
        pltpu.async_copy(w_hbm.at[pl.ds(base + s * CH, CH)],
                         w_v.at[b, s], si[b])

    def wait_idx(b):
      for s in range(sub):
        pltpu.make_async_copy(row_hbm.at[pl.ds(0, CH)], row_v.at[b, s],
                              si[b]).wait()
        pltpu.make_async_copy(col_hbm.at[pl.ds(0, CH)], col_v.at[b, s],
                              si[b]).wait()
        pltpu.make_async_copy(w_hbm.at[pl.ds(0, CH)], w_v.at[b, s],
                              si[b]).wait()

    @pl.loop(0, CH)
    def _(i):
      for d in range(d_feat // 16):
        rows_v[i, pl.ds(d * 16, 16)] = jnp.zeros((16,), jnp.float32)

    @pl.loop(0, CHUNKS_PER_TILE)
    def _(j):
      pltpu.sync_copy(rows_v.at[pl.ds(0, CH)],
                      acc_sh.at[pl.ds(sid * ROWS_PER_TILE + j * CH, CH)])

    plsc.subcore_barrier()

    start_idx(0, 0)
    start_idx(1, 1)

    @pl.loop(0, NCH // 2)
    def _(j):
      for b in (0, 1):
        k = 2 * j + b
        wait_idx(b)

        hg = [pltpu.async_copy(src.at[row_v.at[b, s]],
                               rows_v.at[pl.ds(s * CH, CH)], sg[s])
              for s in range(sub)]
        hs = []
        for s in range(sub):
          hg[s].wait()

          @plsc.parallel_loop(0, CH, unroll=8)
          def _(i):
            w = plsc.load_gather(w_v.at[b, s], [jnp.full((16,), i, jnp.int32)])
            for d in range(d_feat // 16):
              sl = (s * CH + i, pl.ds(d * 16, 16))
              rows_v[sl] = rows_v[sl] * w

          hs.append(pltpu.async_copy(rows_v.at[pl.ds(s * CH, CH)],
                                     acc_sh.at[col_v.at[b, s]], ss[s],
                                     add=True))
        for h in hs:
          h.wait()

        nk = jnp.where(k + 2 >= NCH, k + 2 - NCH, k + 2)
        start_idx(nk, b)

    wait_idx(0)
    wait_idx(1)

    plsc.subcore_barrier()

    @pl.loop(0, CHUNKS_PER_TILE)
    def _(j):
      start = sid * ROWS_PER_TILE + j * CH
      pltpu.sync_copy(acc_sh.at[pl.ds(start, CH)],
                      out_hbm.at[cid, pl.ds(start, CH)])

  return k


_DEG_SUB = 4


@functools.partial(
    pl.kernel,
    out_type=jax.ShapeDtypeStruct((2, NP), jnp.float32),
    mesh=_MESH,
    compiler_params=_SC_PARAMS,
    scratch_types=[
        pltpu.VMEM((2, _DEG_SUB, CH), jnp.int32),
        pltpu.VMEM((2, _DEG_SUB, CH), jnp.float32),
        pltpu.VMEM_SHARED((NP,), jnp.float32),
        pltpu.SemaphoreType.DMA,
        pltpu.SemaphoreType.DMA,
    ],
)
def _degree_kernel(col_hbm, w_hbm, out_hbm, col_v, w_v, deg_sh, si0, si1):
  cid = lax.axis_index("c")
  sid = lax.axis_index("s")
  wid = cid * 16 + sid
  si = (si0, si1)
  chunk = _DEG_SUB * CH
  nch = EPW // chunk

  def start_idx(k, b):
    base = wid * EPW + k * chunk
    for s in range(_DEG_SUB):
      pltpu.async_copy(col_hbm.at[pl.ds(base + s * CH, CH)], col_v.at[b, s],
                       si[b])
      pltpu.async_copy(w_hbm.at[pl.ds(base + s * CH, CH)], w_v.at[b, s],
                       si[b])

  def wait_idx(b):
    for s in range(_DEG_SUB):
      pltpu.make_async_copy(col_hbm.at[pl.ds(0, CH)], col_v.at[b, s],
                            si[b]).wait()
      pltpu.make_async_copy(w_hbm.at[pl.ds(0, CH)], w_v.at[b, s],
                            si[b]).wait()

  @pl.loop(0, CH // 16)
  def _(g):
    w_v[0, 0, pl.ds(g * 16, 16)] = jnp.zeros((16,), jnp.float32)

  @pl.loop(0, CHUNKS_PER_TILE)
  def _(j):
    pltpu.sync_copy(w_v.at[0, 0],
                    deg_sh.at[pl.ds(sid * ROWS_PER_TILE + j * CH, CH)])

  plsc.subcore_barrier()
  start_idx(0, 0)
  start_idx(1, 1)

  @pl.loop(0, nch // 2)
  def _(j):
    for b in (0, 1):
      k = 2 * j + b
      wait_idx(b)
      for s in range(_DEG_SUB):
        pltpu.sync_copy(w_v.at[b, s], deg_sh.at[col_v.at[b, s]], add=True)
      nk = jnp.where(k + 2 >= nch, k + 2 - nch, k + 2)
      start_idx(nk, b)

  wait_idx(0)
  wait_idx(1)
  plsc.subcore_barrier()

  @pl.loop(0, CHUNKS_PER_TILE)
  def _(j):
    start = sid * ROWS_PER_TILE + j * CH
    pltpu.sync_copy(deg_sh.at[pl.ds(start, CH)], out_hbm.at[cid, pl.ds(start, CH)])


_GAT_SUB = 4
_GAT_CHUNK = _GAT_SUB * CH


@functools.partial(
    pl.kernel,
    out_type=[
        jax.ShapeDtypeStruct((2, NP), jnp.float32),
        jax.ShapeDtypeStruct((2, NP, 64), jnp.float32),
    ],
    mesh=_MESH,
    compiler_params=_SC_PARAMS_LINEAR,
    scratch_types=[
        pltpu.VMEM((2, _GAT_SUB, CH), jnp.int32),
        pltpu.VMEM((2, _GAT_SUB, CH), jnp.int32),
        pltpu.VMEM((_GAT_CHUNK,), jnp.float32),
        pltpu.VMEM((_GAT_CHUNK, 2), jnp.float32),
        pltpu.VMEM((_GAT_CHUNK,), jnp.float32),
        pltpu.VMEM((_GAT_CHUNK, 64), jnp.float32),
        pltpu.VMEM_SHARED((NP,), jnp.float32),
        pltpu.VMEM_SHARED((NP, 64), jnp.float32),
        pltpu.VMEM_SHARED((NP, 64), jnp.float32),
        pltpu.VMEM_SHARED((NP,), jnp.float32),
        pltpu.VMEM_SHARED((NP, 2), jnp.float32),
        pltpu.SemaphoreType.DMA,
        pltpu.SemaphoreType.DMA,
    ] + [pltpu.SemaphoreType.DMA] * (5 * _GAT_SUB),
)
def _gat_edge_kernel(h3_hbm, asrc_hbm, pq_hbm, row_hbm, col_hbm,
                     s_out, acc_out, row_v, col_v, as_v, pq_v, ex_v, rows_v,
                     s_sh, acc_sh, h3_sh, as_sh, pq_sh, si0, si1, *sems):
  cid = lax.axis_index("c")
  sid = lax.axis_index("s")
  wid = cid * 16 + sid
  si = (si0, si1)
  nch = EPW // _GAT_CHUNK

  def start_idx(k, b):
    base = wid * EPW + k * _GAT_CHUNK
    for s in range(_GAT_SUB):
      pltpu.async_copy(row_hbm.at[pl.ds(base + s * CH, CH)], row_v.at[b, s],
                       si[b])
      pltpu.async_copy(col_hbm.at[pl.ds(base + s * CH, CH)], col_v.at[b, s],
                       si[b])

  def wait_idx(b):
    for s in range(_GAT_SUB):
      pltpu.make_async_copy(row_hbm.at[pl.ds(0, CH)], row_v.at[b, s],
                            si[b]).wait()
      pltpu.make_async_copy(col_hbm.at[pl.ds(0, CH)], col_v.at[b, s],
                            si[b]).wait()

  @pl.loop(0, CH)
  def _(i):
    for d in range(4):
      rows_v[i, pl.ds(d * 16, 16)] = jnp.zeros((16,), jnp.float32)

  @pl.loop(0, CH // 16)
  def _(g):
    ex_v[pl.ds(g * 16, 16)] = jnp.zeros((16,), jnp.float32)

  @pl.loop(0, CHUNKS_PER_TILE)
  def _(j):
    start = sid * ROWS_PER_TILE + j * CH
    pltpu.sync_copy(rows_v.at[pl.ds(0, CH)], acc_sh.at[pl.ds(start, CH)])
    pltpu.sync_copy(ex_v.at[pl.ds(0, CH)], s_sh.at[pl.ds(start, CH)])

  tile = pl.ds(sid * ROWS_PER_TILE, ROWS_PER_TILE)
  pltpu.sync_copy(h3_hbm.at[tile], h3_sh.at[tile])
  pltpu.sync_copy(asrc_hbm.at[tile], as_sh.at[tile])
  pltpu.sync_copy(pq_hbm.at[tile], pq_sh.at[tile])

  plsc.subcore_barrier()
  start_idx(0, 0)
  start_idx(1, 1)

  @pl.loop(0, nch // 2)
  def _(j):
    for b in (0, 1):
      k = 2 * j + b
      wait_idx(b)

      sa = sems[:_GAT_SUB]
      sp = sems[_GAT_SUB:2 * _GAT_SUB]
      sh = sems[2 * _GAT_SUB:3 * _GAT_SUB]
      se = sems[3 * _GAT_SUB:4 * _GAT_SUB]
      sr = sems[4 * _GAT_SUB:5 * _GAT_SUB]
      ha = [pltpu.async_copy(as_sh.at[row_v.at[b, s]],
                             as_v.at[pl.ds(s * CH, CH)], sa[s])
            for s in range(_GAT_SUB)]
      hp = [pltpu.async_copy(pq_sh.at[col_v.at[b, s]],
                             pq_v.at[pl.ds(s * CH, CH)], sp[s])
            for s in range(_GAT_SUB)]
      hh = [pltpu.async_copy(h3_sh.at[row_v.at[b, s]],
                             rows_v.at[pl.ds(s * CH, CH)], sh[s])
            for s in range(_GAT_SUB)]
      hw = []
      for s in range(_GAT_SUB):
        ha[s].wait()
        hp[s].wait()

        @plsc.parallel_loop(0, CH // 16, unroll=2)
        def _(g):
          lane = lax.iota(jnp.int32, 16) + (s * CH + g * 16)
          ad = plsc.load_gather(pq_v, [lane, jnp.zeros((16,), jnp.int32)])
          off = plsc.load_gather(pq_v, [lane, jnp.ones((16,), jnp.int32)])
          sl = pl.ds(s * CH + g * 16, 16)
          z = as_v[sl] + ad
          e = jnp.where(z > 0.0, z, 0.2 * z)
          ex_v[sl] = jnp.exp(e - off)

        hw.append(pltpu.async_copy(ex_v.at[pl.ds(s * CH, CH)],
                                   s_sh.at[col_v.at[b, s]], se[s], add=True))
      for s in range(_GAT_SUB):
        hh[s].wait()

        @plsc.parallel_loop(0, CH, unroll=8)
        def _(i):
          ii = s * CH + i
          w = plsc.load_gather(ex_v, [jnp.full((16,), ii, jnp.int32)])
          for d in range(4):
            sl = (ii, pl.ds(d * 16, 16))
            rows_v[sl] = rows_v[sl] * w

        hw.append(pltpu.async_copy(rows_v.at[pl.ds(s * CH, CH)],
                                   acc_sh.at[col_v.at[b, s]], sr[s], add=True))
      for h in hw:
        h.wait()

      nk = jnp.where(k + 2 >= nch, k + 2 - nch, k + 2)
      start_idx(nk, b)

  wait_idx(0)
  wait_idx(1)
  plsc.subcore_barrier()

  @pl.loop(0, CHUNKS_PER_TILE)
  def _(j):
    start = sid * ROWS_PER_TILE + j * CH
    pltpu.sync_copy(s_sh.at[pl.ds(start, CH)], s_out.at[cid, pl.ds(start, CH)])
    pltpu.sync_copy(acc_sh.at[pl.ds(start, CH)],
                    acc_out.at[cid, pl.ds(start, CH)])


BR = 2000
GRID = N_NODES // BR


def _rb(d):
  return pl.BlockSpec((BR, d), lambda i: (i, 0))


def _full(s0, s1):
  return pl.BlockSpec((s0, s1), lambda i: (0, 0))


def _part(core, d):
  return pl.BlockSpec((1, BR, d), lambda i, c=core: (c, i, 0))


def _row_call(body, in_specs, out_shape, out_specs):
  return pl.pallas_call(body, grid=(GRID,), in_specs=in_specs,
                        out_shape=out_shape, out_specs=out_specs)


_DOT = functools.partial(jnp.dot, preferred_element_type=jnp.float32,
                         precision=lax.Precision.HIGHEST)


def _mm_kernel(x_ref, w_ref, o_ref):
  o_ref[...] = _DOT(x_ref[...], w_ref[...])


def _scale_kernel(hp_ref, d0_ref, d1_ref, g_ref, dinv_ref):
  deg = d0_ref[0] + d1_ref[0] + 1.0
  dinv = lax.rsqrt(deg)
  dinv_ref[...] = dinv
  g_ref[...] = hp_ref[...] * dinv


def _combine_kernel(a0_ref, a1_ref, hp_ref, dinv_ref, b_ref, w_ref,
                    hnext_ref, gnext_ref):
  dinv = dinv_ref[...]
  out = dinv * (a0_ref[0] + a1_ref[0]) + dinv * dinv * hp_ref[...] \
      + b_ref[...]
  h = out * (1.0 / (1.0 + jnp.exp(-out)))
  hp = _DOT(h, w_ref[...])
  hnext_ref[...] = hp
  gnext_ref[...] = hp * dinv


def _gat_mm_kernel(a0_ref, a1_ref, hp_ref, dinv_ref, b_ref, w_ref,
                   atts_ref, attd_ref, h3_ref, asrc_ref, adst_ref):
  dinv = dinv_ref[...]
  out = dinv * (a0_ref[0] + a1_ref[0]) + dinv * dinv * hp_ref[...] \
      + b_ref[...]
  h2 = out * (1.0 / (1.0 + jnp.exp(-out)))
  h3 = _DOT(h2, w_ref[...])
  h3_ref[...] = h3
  asrc_ref[...] = _DOT(h3, atts_ref[...])
  adst_ref[...] = _DOT(h3, attd_ref[...])


def _att_prep_kernel(asrc_ref, adst_ref, pq_ref, exs_ref):
  asrc = asrc_ref[...]
  adst = adst_ref[...]
  amax = jnp.max(asrc)
  es = asrc + adst
  e_self = jnp.where(es > 0.0, es, 0.2 * es)
  off = jnp.maximum(e_self, adst + amax)
  pq_ref[...] = jnp.concatenate([adst, off], axis=1)
  exs_ref[...] = jnp.exp(e_self - off)


def _final_kernel(a0_ref, a1_ref, s0_ref, s1_ref, exs_ref, h3_ref, b_ref,
                  o_ref):
  s = s0_ref[0] + s1_ref[0] + exs_ref[...]
  num = a0_ref[0] + a1_ref[0] + exs_ref[...] * h3_ref[...]
  o3 = num / s + b_ref[...]
  m = jnp.max(o3, axis=1, keepdims=True)
  e = jnp.exp(o3 - m)
  o_ref[...] = e / jnp.sum(e, axis=1, keepdims=True)


def kernel(x, edge_index, edge_weight, W1, b1, W2, b2, W3, att_src, att_dst,
           b3):
  n = N_NODES
  row, col = edge_index[0], edge_index[1]

  pad = EPAD - N_EDGES
  pad_idx = (jnp.arange(pad, dtype=jnp.int32) * 8) % n
  row_p = jnp.concatenate([row, pad_idx])
  col_p = jnp.concatenate([col, pad_idx])
  ew_p = jnp.concatenate([edge_weight, jnp.zeros((pad,), jnp.float32)])
  row_q = jnp.concatenate(
      [row, n + (jnp.arange(pad, dtype=jnp.int32) % 16)])

  deg_parts = _degree_kernel(col_p, ew_p)
  h1p = _row_call(_mm_kernel, [_rb(128), _full(128, 128)],
                  jax.ShapeDtypeStruct((n, 128), jnp.float32),
                  _rb(128))(x, W1)

  dp = deg_parts.reshape(2, NP, 1)
  g1, dinv = _row_call(
      _scale_kernel, [_rb(128), _part(0, 1), _part(1, 1)],
      (jax.ShapeDtypeStruct((n, 128), jnp.float32),
       jax.ShapeDtypeStruct((n, 1), jnp.float32)),
      (_rb(128), _rb(1)))(h1p, dp, dp)

  acc1 = _edge_accumulate(128, 2, False)(g1, row_p, col_p, ew_p)
  h2p, g2 = _row_call(
      _combine_kernel,
      [_part(0, 128), _part(1, 128), _rb(128), _rb(1), _full(1, 128),
       _full(128, 64)],
      (jax.ShapeDtypeStruct((n, 64), jnp.float32),
       jax.ShapeDtypeStruct((n, 64), jnp.float32)),
      (_rb(64), _rb(64)))(
          acc1, acc1, h1p, dinv, b1.reshape(1, 128), W2)

  g2x = jnp.concatenate([g2, jnp.zeros((NP - n, 64), jnp.float32)])
  acc2 = _edge_accumulate(64, 4, True)(g2x, row_p, col_p, ew_p)
  h3, asrc, adst = _row_call(
      _gat_mm_kernel,
      [_part(0, 64), _part(1, 64), _rb(64), _rb(1), _full(1, 64),
       _full(64, 64), _full(64, 1), _full(64, 1)],
      (jax.ShapeDtypeStruct((n, 64), jnp.float32),
       jax.ShapeDtypeStruct((n, 1), jnp.float32),
       jax.ShapeDtypeStruct((n, 1), jnp.float32)),
      (_rb(64), _rb(1), _rb(1)))(
          acc2, acc2, h2p, dinv, b2.reshape(1, 64), W3,
          att_src.reshape(64, 1), att_dst.reshape(64, 1))

  pq, exs = pl.pallas_call(
      _att_prep_kernel,
      out_shape=(jax.ShapeDtypeStruct((n, 2), jnp.float32),
                 jax.ShapeDtypeStruct((n, 1), jnp.float32)))(asrc, adst)

  asx = jnp.concatenate(
      [asrc.reshape(n), jnp.full((NP - n,), -1e30, jnp.float32)])
  h3x = jnp.concatenate([h3, jnp.zeros((NP - n, 64), jnp.float32)])
  pqx = jnp.concatenate([pq, jnp.zeros((NP - n, 2), jnp.float32)])
  s_parts, acc3 = _gat_edge_kernel(h3x, asx, pqx, row_q, col_p)

  sp = s_parts.reshape(2, NP, 1)
  out = _row_call(
      _final_kernel,
      [_part(0, 64), _part(1, 64), _part(0, 1), _part(1, 1), _rb(1), _rb(64),
       _full(1, 64)],
      jax.ShapeDtypeStruct((n, 64), jnp.float32),
      _rb(64))(
          acc3, acc3, sp, sp, exs, h3, b3.reshape(1, 64))
  return out
